# baseline (speedup 1.0000x reference)
import functools
import math

import numpy as np
import jax
import jax.numpy as jnp
from jax import lax
from jax.experimental import pallas as pl
from jax.experimental.pallas import tpu as pltpu
from jax.experimental.pallas import tpu_sc as plsc

HI = None

_BMG = 256
_NW = 32
_NPAIR = 8192
_CHUNK = _NPAIR // _NW
_NB = (_NPAIR + 5 * (_BMG - 1) + _BMG - 1) // _BMG
_PT = _NB * _BMG

_EXPERT_TYPES = ['conv', 'fourier', 'mlp', 'conv', 'fourier', 'mlp', 'conv', 'fourier']
_SPARSE_EIDS = [0, 2, 3, 5, 6]
_FOURIER_EIDS = [1, 4, 7]
_JMAP = [0, 5, 1, 2, 6, 3, 4, 7]


def _gelu(h):
    return h * 0.5 * (1.0 + jax.lax.erf(h / np.float32(np.sqrt(2.0))))


def _router_body(xf, wp, bp, w01_ref, jp_ref, aux_ref, *, E, topk):
    x = xf[...]
    logits = jax.lax.dot_general(x, wp[...], (((1,), (0,)), ((), ())),
                                 preferred_element_type=jnp.float32) + bp[...]
    lanes = jax.lax.broadcasted_iota(jnp.int32, logits.shape, 1)
    neg = jnp.float32(-1e30)
    logits = jnp.where(lanes < E, logits, neg)
    m = jnp.max(logits, axis=-1, keepdims=True)
    ex = jnp.where(lanes < E, jnp.exp(logits - m), 0.0)
    probs = ex / jnp.sum(ex, axis=-1, keepdims=True)
    m1 = jnp.max(probs, axis=-1, keepdims=True)
    a1 = jnp.min(jnp.where(probs >= m1, lanes, E), axis=-1, keepdims=True)
    p2 = jnp.where(lanes == a1, neg, probs)
    m2 = jnp.max(p2, axis=-1, keepdims=True)
    a2 = jnp.min(jnp.where(p2 >= m2, lanes, E), axis=-1, keepdims=True)
    denom = m1 + m2
    w0 = m1 / denom
    w1 = m2 / denom
    col = lanes
    w01_ref[...] = jnp.where(col == 0, w0, 0.0) + jnp.where(col == 1, w1, 0.0)
    j1 = jnp.zeros_like(a1)
    j2 = jnp.zeros_like(a2)
    for e in range(E):
        j1 = j1 + jnp.where(a1 == e, _JMAP[e], 0)
        j2 = j2 + jnp.where(a2 == e, _JMAP[e], 0)
    jp_ref[...] = (jnp.where(col == 0, j1, 0) + jnp.where(col == 1, j2, 0)
                   ).astype(jnp.int32)
    ntok = x.shape[0]
    me = jnp.sum(probs, axis=0) / ntok
    cnt = jnp.sum((lanes == a1).astype(jnp.float32)
                  + (lanes == a2).astype(jnp.float32), axis=0)
    ce = cnt / (ntok * topk)
    aux_ref[...] = jnp.reshape(E * jnp.sum(me * ce), (1, 1))


def _router_call(xf, router_w, router_b, E, topk):
    M, D = xf.shape
    wp = jnp.zeros((D, 128), jnp.float32).at[:, :E].set(router_w)
    bp = jnp.zeros((1, 128), jnp.float32).at[0, :E].set(router_b)
    out = pl.pallas_call(
        functools.partial(_router_body, E=E, topk=topk),
        out_shape=(jax.ShapeDtypeStruct((M, 128), jnp.float32),
                   jax.ShapeDtypeStruct((M, 128), jnp.int32),
                   jax.ShapeDtypeStruct((1, 1), jnp.float32)),
    )(xf, wp, bp)
    return out


def _convfields_body(xm_ref, xb_ref, xp_ref, cw_ref, tbl_ref, *, nsb):
    i = pl.program_id(1)
    xb = xb_ref[0]
    prev = jnp.concatenate([xm_ref[0, -1:], xb[:-1]], axis=0)
    nxt = jnp.concatenate([xb[1:], xp_ref[0, :1]], axis=0)
    rows = jax.lax.broadcasted_iota(jnp.int32, xb.shape, 0)
    prev = jnp.where((i == 0) & (rows == 0), 0.0, prev)
    nxt = jnp.where((i == nsb - 1) & (rows == xb.shape[0] - 1), 0.0, nxt)
    tbl_ref[0, 0] = xb.astype(jnp.bfloat16)
    for t in range(3):
        c = (prev * cw_ref[t, 0][None, :] + xb * cw_ref[t, 1][None, :]
             + nxt * cw_ref[t, 2][None, :])
        tbl_ref[t + 1, 0] = (xb + c).astype(jnp.bfloat16)


def _convfields_call(x, conv_ws):
    B, S, D = x.shape
    BS = min(512, S)
    nsb = S // BS
    cw = jnp.stack([jnp.transpose(w[:, 0, :], (1, 0)) for w in conv_ws])
    grid = (B, nsb)
    out = pl.pallas_call(
        functools.partial(_convfields_body, nsb=nsb),
        grid=grid,
        in_specs=[
            pl.BlockSpec((1, BS, D), lambda b, i: (b, jnp.maximum(i - 1, 0), 0)),
            pl.BlockSpec((1, BS, D), lambda b, i: (b, i, 0)),
            pl.BlockSpec((1, BS, D), lambda b, i: (b, jnp.minimum(i + 1, nsb - 1), 0)),
            pl.BlockSpec((3, 3, D), lambda b, i: (0, 0, 0)),
        ],
        out_specs=pl.BlockSpec((4, 1, BS, D), lambda b, i: (0, b, i, 0)),
        out_shape=jax.ShapeDtypeStruct((4, B, S, D), jnp.bfloat16),
    )(x, x, x, cw)
    return out.reshape(4, B * S, D)


def _mlp_body(x_ref, w1_ref, b1_ref, w2_ref, b2_ref, gw_ref, gb_ref, o_ref,
              *, nf, gated):
    f = pl.program_id(1)

    @pl.when(f == 0)
    def _():
        o_ref[...] = jnp.broadcast_to(b2_ref[...], o_ref.shape)

    x = x_ref[...]
    h = jax.lax.dot_general(x, w1_ref[...], (((1,), (0,)), ((), ())),
                            preferred_element_type=jnp.float32, precision=HI)
    h = _gelu(h + b1_ref[...])
    o_ref[...] += jax.lax.dot_general(h, w2_ref[...], (((1,), (0,)), ((), ())),
                                      preferred_element_type=jnp.float32,
                                      precision=HI)
    if gated:
        @pl.when(f == nf - 1)
        def _():
            g = jax.lax.dot_general(x, gw_ref[...], (((1,), (0,)), ((), ())),
                                    preferred_element_type=jnp.float32,
                                    precision=HI)
            g = jax.nn.sigmoid(g[:, :1] + gb_ref[0:1, 0:1])
            o_ref[...] *= g


def _mlp_call(xf, w1, b1, w2, b2, gate=None):
    M, D = xf.shape
    F = w1.shape[1]
    N = w2.shape[1]
    BM = min(256, M)
    BF = min(512, F)
    nf = F // BF
    gated = gate is not None
    if gated:
        gw, gb = gate
        gwp = jnp.zeros((D, 128), jnp.float32).at[:, :1].set(gw)
        gbp = jnp.full((1, 1), gb[0], jnp.float32)
    else:
        gwp = jnp.zeros((1, 128), jnp.float32)
        gbp = jnp.zeros((1, 1), jnp.float32)
    return pl.pallas_call(
        functools.partial(_mlp_body, nf=nf, gated=gated),
        grid=(M // BM, nf),
        in_specs=[
            pl.BlockSpec((BM, D), lambda m, f: (m, 0)),
            pl.BlockSpec((D, BF), lambda m, f: (0, f)),
            pl.BlockSpec((1, BF), lambda m, f: (0, f)),
            pl.BlockSpec((BF, N), lambda m, f: (f, 0)),
            pl.BlockSpec((1, N), lambda m, f: (0, 0)),
            pl.BlockSpec(gwp.shape, lambda m, f: (0, 0)),
            pl.BlockSpec((1, 1), lambda m, f: (0, 0)),
        ],
        out_specs=pl.BlockSpec((BM, N), lambda m, f: (m, 0)),
        out_shape=jax.ShapeDtypeStruct((M, N), jnp.float32),
    )(xf, w1, b1.reshape(1, F), w2, b2.reshape(1, N), gwp, gbp)


@functools.lru_cache(maxsize=2)
def _dft_consts(S):
    F = S // 2 + 1
    Fp = ((F + 127) // 128) * 128
    s = np.arange(S)
    f = np.arange(F)
    ang = 2.0 * np.pi * np.outer(f, s) / S
    CS = np.zeros((2 * Fp, S), np.float32)
    CS[:F] = np.cos(ang)
    CS[Fp:Fp + F] = -np.sin(ang)
    cr = np.full(F, 2.0); cr[0] = 1.0; cr[-1] = 1.0
    ci = np.full(F, 2.0); ci[0] = 0.0; ci[-1] = 0.0
    angT = ang.T
    CrCi = np.zeros((2, S, Fp), np.float32)
    CrCi[0, :, :F] = np.cos(angT) * cr / S
    CrCi[1, :, :F] = -np.sin(angT) * ci / S
    return CS, CrCi, Fp


def _matmul_body(a_ref, b_ref, o_ref, *, nk):
    k = pl.program_id(2)

    @pl.when(k == 0)
    def _():
        o_ref[...] = jnp.zeros_like(o_ref)

    o_ref[...] += jax.lax.dot_general(
        a_ref[...], b_ref[0], (((1,), (0,)), ((), ())),
        preferred_element_type=jnp.float32, precision=HI)


def _dft_call(x, CS):
    B, S, D = x.shape
    Fp2 = CS.shape[0]
    BM = min(256, Fp2)
    BK = min(1024, S)
    nk = S // BK
    out = pl.pallas_call(
        functools.partial(_matmul_body, nk=nk),
        grid=(B, Fp2 // BM, nk),
        in_specs=[
            pl.BlockSpec((BM, BK), lambda b, m, k: (m, k)),
            pl.BlockSpec((1, BK, D), lambda b, m, k: (b, k, 0)),
        ],
        out_specs=pl.BlockSpec((1, BM, D), lambda b, m, k: (b, m, 0)),
        out_shape=jax.ShapeDtypeStruct((B, Fp2, D), jnp.float32),
    )(CS, x)
    return out.reshape(B, 2, Fp2 // 2, D)


def _fmlp_body(ri_ref, w1_ref, b1_ref, w2_ref, b2_ref, o_ref, *, D):
    f = pl.program_id(2)

    @pl.when(f == 0)
    def _():
        o_ref[0, 0] = jnp.broadcast_to(b2_ref[:, :D], o_ref.shape[2:])
        o_ref[1, 0] = jnp.broadcast_to(b2_ref[:, D:], o_ref.shape[2:])

    re = ri_ref[0, 0]
    im = ri_ref[0, 1]
    h = jax.lax.dot_general(re, w1_ref[:D], (((1,), (0,)), ((), ())),
                            preferred_element_type=jnp.float32, precision=HI)
    h += jax.lax.dot_general(im, w1_ref[D:], (((1,), (0,)), ((), ())),
                             preferred_element_type=jnp.float32, precision=HI)
    h = _gelu(h + b1_ref[...])
    fo_re = jax.lax.dot_general(h, w2_ref[:, :D], (((1,), (0,)), ((), ())),
                                preferred_element_type=jnp.float32, precision=HI)
    fo_im = jax.lax.dot_general(h, w2_ref[:, D:], (((1,), (0,)), ((), ())),
                                preferred_element_type=jnp.float32, precision=HI)
    o_ref[0, 0] += fo_re
    o_ref[1, 0] += fo_im


def _fmlp_call(RI, w1, b1, w2, b2):
    B, _, Fp, D = RI.shape
    FF = w1.shape[1]
    BM = min(128, Fp)
    BF = min(512, FF)
    return pl.pallas_call(
        functools.partial(_fmlp_body, D=D),
        grid=(B, Fp // BM, FF // BF),
        in_specs=[
            pl.BlockSpec((1, 2, BM, D), lambda b, m, f: (b, 0, m, 0)),
            pl.BlockSpec((2 * D, BF), lambda b, m, f: (0, f)),
            pl.BlockSpec((1, BF), lambda b, m, f: (0, f)),
            pl.BlockSpec((BF, 2 * D), lambda b, m, f: (f, 0)),
            pl.BlockSpec((1, 2 * D), lambda b, m, f: (0, 0)),
        ],
        out_specs=pl.BlockSpec((2, 1, BM, D), lambda b, m, f: (0, b, m, 0)),
        out_shape=jax.ShapeDtypeStruct((2, B, Fp, D), jnp.float32),
    )(RI, w1, b1.reshape(1, FF), w2, b2.reshape(1, 2 * D))


def _irfft_body(c_ref, fo_ref, o_ref):
    p = pl.program_id(2)

    @pl.when(p == 0)
    def _():
        o_ref[...] = jnp.zeros_like(o_ref)

    o_ref[0] += jax.lax.dot_general(c_ref[0], fo_ref[0, 0],
                                    (((1,), (0,)), ((), ())),
                                    preferred_element_type=jnp.float32,
                                    precision=HI)


def _irfft_call(FO, CrCi):
    _, B, Fp, D = FO.shape
    S = CrCi.shape[1]
    BM = min(256, S)
    return pl.pallas_call(
        _irfft_body,
        grid=(B, S // BM, 2),
        in_specs=[
            pl.BlockSpec((1, BM, Fp), lambda b, s, p: (p, s, 0)),
            pl.BlockSpec((1, 1, Fp, D), lambda b, s, p: (p, b, 0, 0)),
        ],
        out_specs=pl.BlockSpec((1, BM, D), lambda b, s, p: (b, s, 0)),
        out_shape=jax.ShapeDtypeStruct((B, S, D), jnp.float32),
    )(CrCi, FO)


_FLD = [1, 0, 2, 0, 3]


def _dispatch_body(jflat, table, g_hbm, pos_hbm, bexp_hbm, bact_hbm,
                   eidv, gidx, sidx, pidx, buf0, buf1, buf2, bescr, bascr,
                   sem0, sem1, sem2, *, ntok):
    nc = 2
    wid = lax.axis_index("s") * nc + lax.axis_index("c")
    pltpu.sync_copy(jflat, eidv)
    lanes = lax.iota(jnp.int32, 16)

    def count_step(i, accs):
        v = eidv[pl.ds(i * 16, 16)]
        return tuple(a + jnp.where(v == j, 1, 0)
                     for j, a in enumerate(accs))

    def reduce_accs(accs):
        tot = jnp.zeros(16, jnp.int32)
        for j in range(5):
            tot = tot + jnp.where(lanes == j, jnp.sum(accs[j]), 0)
        return tot

    z5 = tuple(jnp.zeros(16, jnp.int32) for _ in range(5))
    totals = reduce_accs(lax.fori_loop(0, _NPAIR // 16, count_step, z5))
    pre = reduce_accs(lax.fori_loop(0, (_CHUNK // 16) * wid, count_step, z5))
    asz = ((totals + (_BMG - 1)) >> 8) << 8
    starts = plsc.cumsum(asz) - asz
    cur = starts + pre

    for v in range(_CHUNK // 16):
        jv = eidv[pl.ds(wid * _CHUNK + v * 16, 16)]
        qv = wid * _CHUNK + v * 16 + lanes
        tv = qv - jnp.where(qv >= ntok, ntok, 0)
        sp = jv < 5
        rank = jnp.zeros(16, jnp.int32)
        basev = jnp.zeros(16, jnp.int32)
        fldv = jnp.zeros(16, jnp.int32)
        for j in range(5):
            mj = jv == j
            cs = plsc.cumsum(mj.astype(jnp.int32))
            rank = rank + jnp.where(mj, cs - 1, 0)
            cj = jnp.sum(jnp.where(lanes == j, cur, 0))
            basev = jnp.where(mj, cj, basev)
            fldv = jnp.where(mj, _FLD[j], fldv)
        slot = basev + rank
        pidx[v // 4, pl.ds((v % 4) * 16, 16)] = jnp.where(sp, slot, 0)
        sidx[v // 4, pl.ds((v % 4) * 16, 16)] = jnp.where(sp, slot, _PT)
        gidx[v // 4, pl.ds((v % 4) * 16, 16)] = jnp.where(
            sp, fldv * ntok + tv, 0)
        for j in range(5):
            c = jnp.sum(jnp.where(jv == j, 1, 0))
            cur = cur + jnp.where(lanes == j, c, 0)

    nchunk = _CHUNK // 64
    bufs = [buf0, buf1, buf2]
    sems = [sem0, sem1, sem2]
    gcp = {}
    scp = {}
    for c in range(3):
        gcp[c] = pltpu.async_copy(table.at[gidx.at[c]], bufs[c], sems[c])
    for c in range(nchunk):
        i = c % 3
        gcp[c].wait()
        scp[c] = pltpu.async_copy(bufs[i], g_hbm.at[sidx.at[c]], sems[i])
        if c + 3 < nchunk:
            scp[c].wait()
            gcp[c + 3] = pltpu.async_copy(table.at[gidx.at[c + 3]],
                                          bufs[i], sems[i])
    for c in range(max(nchunk - 3, 0), nchunk):
        scp[c].wait()

    pltpu.sync_copy(pidx, pos_hbm.at[wid])

    @pl.when(wid == 0)
    def _():
        for g4 in range(4):
            bv = g4 * 16 + lanes
            fr = bv << 8
            be = jnp.zeros(16, jnp.int32)
            ba = jnp.zeros(16, jnp.int32)
            for j in range(5):
                sj = jnp.sum(jnp.where(lanes == j, starts, 0))
                aj = jnp.sum(jnp.where(lanes == j, asz, 0))
                tj = jnp.sum(jnp.where(lanes == j, totals, 0))
                inj = (fr >= sj) & (fr < sj + aj)
                be = jnp.where(inj, j, be)
                ba = jnp.where(inj & (fr < sj + tj), 1, ba)
            bescr[pl.ds(g4 * 16, 16)] = be
            bascr[pl.ds(g4 * 16, 16)] = ba
        pltpu.sync_copy(bescr, bexp_hbm)
        pltpu.sync_copy(bascr, bact_hbm)


def _dispatch_call(jflat, table):
    ntok = table.shape[0] // 4
    D = table.shape[1]
    mesh = plsc.VectorSubcoreMesh(core_axis_name="c", subcore_axis_name="s")
    fn = functools.partial(
        pl.kernel,
        mesh=mesh,
        compiler_params=pltpu.CompilerParams(needs_layout_passes=False),
        out_type=(jax.ShapeDtypeStruct((_PT + 8, D), jnp.int32),
                  jax.ShapeDtypeStruct((_NW, _CHUNK // 64, 64), jnp.int32),
                  jax.ShapeDtypeStruct((64,), jnp.int32),
                  jax.ShapeDtypeStruct((64,), jnp.int32)),
        scratch_types=[
            pltpu.VMEM((_NPAIR,), jnp.int32),
            pltpu.VMEM((_CHUNK // 64, 64), jnp.int32),
            pltpu.VMEM((_CHUNK // 64, 64), jnp.int32),
            pltpu.VMEM((_CHUNK // 64, 64), jnp.int32),
            pltpu.VMEM((64, D), jnp.int32),
            pltpu.VMEM((64, D), jnp.int32),
            pltpu.VMEM((64, D), jnp.int32),
            pltpu.VMEM((64,), jnp.int32),
            pltpu.VMEM((64,), jnp.int32),
            pltpu.SemaphoreType.DMA,
            pltpu.SemaphoreType.DMA,
            pltpu.SemaphoreType.DMA,
        ],
    )(functools.partial(_dispatch_body, ntok=ntok))
    return fn(jflat, table)


def _cgather_body(y_hbm, pos_hbm, out_hbm, pidx, buf0, buf1, buf2,
                  sem0, sem1, sem2):
    nc = 2
    wid = lax.axis_index("s") * nc + lax.axis_index("c")
    pltpu.sync_copy(pos_hbm.at[wid], pidx)
    nchunk = _CHUNK // 64
    bufs = [buf0, buf1, buf2]
    sems = [sem0, sem1, sem2]
    gcp = {}
    scp = {}
    for c in range(min(3, nchunk)):
        gcp[c] = pltpu.async_copy(y_hbm.at[pidx.at[c]], bufs[c], sems[c])
    for c in range(nchunk):
        i = c % 3
        gcp[c].wait()
        scp[c] = pltpu.async_copy(
            bufs[i], out_hbm.at[pl.ds(wid * _CHUNK + c * 64, 64)], sems[i])
        if c + 3 < nchunk:
            scp[c].wait()
            gcp[c + 3] = pltpu.async_copy(y_hbm.at[pidx.at[c + 3]],
                                          bufs[i], sems[i])
    for c in range(max(nchunk - 3, 0), nchunk):
        scp[c].wait()


def _cgather_call(Y, pos):
    D = Y.shape[1]
    mesh = plsc.VectorSubcoreMesh(core_axis_name="c", subcore_axis_name="s")
    fn = functools.partial(
        pl.kernel,
        mesh=mesh,
        compiler_params=pltpu.CompilerParams(needs_layout_passes=False),
        out_type=jax.ShapeDtypeStruct((_NPAIR, D), jnp.int32),
        scratch_types=[
            pltpu.VMEM((_CHUNK // 64, 64), jnp.int32),
            pltpu.VMEM((64, D), jnp.int32),
            pltpu.VMEM((64, D), jnp.int32),
            pltpu.VMEM((64, D), jnp.int32),
            pltpu.SemaphoreType.DMA,
            pltpu.SemaphoreType.DMA,
            pltpu.SemaphoreType.DMA,
        ],
    )(_cgather_body)
    return fn(Y, pos)


def _gmlp_body(bexp_ref, bact_ref, g_ref, w1_ref, b1_ref, w2_ref, b2_ref,
               y_ref):
    b = pl.program_id(0)

    @pl.when(bact_ref[b] == 1)
    def _():
        xg = g_ref[...]
        h = jax.lax.dot_general(xg, w1_ref[0], (((1,), (0,)), ((), ())),
                                preferred_element_type=jnp.float32,
                                precision=HI)
        h = _gelu(h + b1_ref[0]).astype(jnp.bfloat16)
        y = jax.lax.dot_general(h, w2_ref[0], (((1,), (0,)), ((), ())),
                                preferred_element_type=jnp.float32,
                                precision=HI)
        y_ref[...] = (y + b2_ref[0]).astype(jnp.bfloat16)


def _gmlp_call(G, w1s, b1s, w2s, b2s, bexp, bact):
    D = G.shape[1]
    FF = w1s.shape[2]
    grid_spec = pltpu.PrefetchScalarGridSpec(
        num_scalar_prefetch=2,
        grid=(_NB,),
        in_specs=[
            pl.BlockSpec((_BMG, D), lambda b, be, ba: (b, 0)),
            pl.BlockSpec((1, D, FF), lambda b, be, ba: (be[b], 0, 0)),
            pl.BlockSpec((1, 1, FF), lambda b, be, ba: (be[b], 0, 0)),
            pl.BlockSpec((1, FF, D), lambda b, be, ba: (be[b], 0, 0)),
            pl.BlockSpec((1, 1, D), lambda b, be, ba: (be[b], 0, 0)),
        ],
        out_specs=pl.BlockSpec((_BMG, D), lambda b, be, ba: (b, 0)),
    )
    return pl.pallas_call(
        _gmlp_body,
        grid_spec=grid_spec,
        out_shape=jax.ShapeDtypeStruct((_PT, D), jnp.bfloat16),
    )(bexp, bact, G, w1s, b1s, w2s, b2s)


def _combine_body(base_ref, w_ref, j_ref, g0_ref, g1_ref, *rest):
    eo_refs = rest[:-1]
    o_ref = rest[-1]
    w0 = w_ref[:, 0:1]
    w1 = w_ref[:, 1:2]
    j0 = j_ref[:, 0:1]
    j1 = j_ref[:, 1:2]
    acc = base_ref[...]
    acc = acc + jnp.where(j0 < 5, w0 * g0_ref[...], 0.0)
    acc = acc + jnp.where(j1 < 5, w1 * g1_ref[...], 0.0)
    for jf, eo in enumerate(eo_refs):
        j = 5 + jf
        coef = (jnp.where(j0 == j, w0, 0.0) + jnp.where(j1 == j, w1, 0.0))
        acc = acc + coef * eo[...]
    o_ref[...] = acc


def _combine_call(base, w01, jp, garr, eos):
    M, D = base.shape
    BM = 256
    off = M // BM
    nspec = [pl.BlockSpec((BM, D), lambda m: (m, 0)),
             pl.BlockSpec((BM, 128), lambda m: (m, 0)),
             pl.BlockSpec((BM, 128), lambda m: (m, 0)),
             pl.BlockSpec((BM, D), lambda m: (m, 0)),
             pl.BlockSpec((BM, D), lambda m: (m + off, 0))]
    nspec += [pl.BlockSpec((BM, D), lambda m: (m, 0)) for _ in eos]
    return pl.pallas_call(
        _combine_body,
        grid=(M // BM,),
        in_specs=nspec,
        out_specs=pl.BlockSpec((BM, D), lambda m: (m, 0)),
        out_shape=jax.ShapeDtypeStruct((M, D), jnp.float32),
    )(base, w01, jp, garr, garr, *eos)


def kernel(x, params):
    B, S, D = x.shape
    E = params['router_b'].shape[0]
    xf = x.reshape(B * S, D)

    w01, jp, aux = _router_call(xf, params['router_w'], params['router_b'],
                                E, 2)

    conv_ws = [params['experts'][e]['conv_w'] for e in _SPARSE_EIDS
               if _EXPERT_TYPES[e] == 'conv']
    table = _convfields_call(x, conv_ws)

    jflat = jnp.concatenate([jp[:, 0], jp[:, 1]]).astype(jnp.int32)
    tbl32 = lax.bitcast_convert_type(
        table.reshape(4 * B * S, D // 2, 2), jnp.int32)
    G32, pos, bexp, bact = _dispatch_call(jflat, tbl32)
    G = lax.bitcast_convert_type(G32, jnp.bfloat16).reshape(_PT + 8, D)

    base = _mlp_call(xf, params['shared_w1'], params['shared_b1'],
                     params['shared_w2'], params['shared_b2'],
                     gate=(params['gate_w'], params['gate_b']))

    CS_np, CrCi_np, Fp = _dft_consts(S)
    CS = jnp.asarray(CS_np)
    CrCi = jnp.asarray(CrCi_np)
    RI = _dft_call(x, CS)

    w1s = jnp.stack([params['experts'][e]['w1'] for e in _SPARSE_EIDS]
                    ).astype(jnp.bfloat16)
    b1s = jnp.stack([params['experts'][e]['b1'].reshape(1, -1)
                     for e in _SPARSE_EIDS])
    w2s = jnp.stack([params['experts'][e]['w2'] for e in _SPARSE_EIDS]
                    ).astype(jnp.bfloat16)
    b2s = jnp.stack([params['experts'][e]['b2'].reshape(1, -1)
                     for e in _SPARSE_EIDS])
    Y = _gmlp_call(G[:_PT], w1s, b1s, w2s, b2s, bexp, bact)
    Y32 = lax.bitcast_convert_type(Y.reshape(_PT, D // 2, 2), jnp.int32)
    garr32 = _cgather_call(Y32, pos)
    garr = lax.bitcast_convert_type(garr32, jnp.bfloat16).reshape(
        _NPAIR, D)

    f_eos = []
    for e in _FOURIER_EIDS:
        p = params['experts'][e]
        FO = _fmlp_call(RI, p['w1'], p['b1'], p['w2'], p['b2'])
        f_eos.append(_irfft_call(FO, CrCi).reshape(B * S, D))

    out = _combine_call(base, w01, jp, garr, f_eos)
    return out.reshape(B, S, D), aux[0, 0]


# speedup vs baseline: 1.9323x; 1.9323x over previous
import functools
import math

import numpy as np
import jax
import jax.numpy as jnp
from jax import lax
from jax.experimental import pallas as pl
from jax.experimental.pallas import tpu as pltpu
from jax.experimental.pallas import tpu_sc as plsc

HI = None

_BMG = 256
_NW = 32
_NPAIR = 8192
_CHUNK = _NPAIR // _NW
_NB = (_NPAIR + 5 * (_BMG - 1) + _BMG - 1) // _BMG
_PT = _NB * _BMG

_EXPERT_TYPES = ['conv', 'fourier', 'mlp', 'conv', 'fourier', 'mlp', 'conv', 'fourier']
_SPARSE_EIDS = [0, 2, 3, 5, 6]
_FOURIER_EIDS = [1, 4, 7]
_JMAP = [0, 5, 1, 2, 6, 3, 4, 7]


def _gelu(h):
    return h * 0.5 * (1.0 + jax.lax.erf(h / np.float32(np.sqrt(2.0))))


def _router_body(xf, wp, bp, w01_ref, jp_ref, aux_ref, *, E, topk):
    x = xf[...]
    logits = jax.lax.dot_general(x, wp[...], (((1,), (0,)), ((), ())),
                                 preferred_element_type=jnp.float32) + bp[...]
    lanes = jax.lax.broadcasted_iota(jnp.int32, logits.shape, 1)
    neg = jnp.float32(-1e30)
    logits = jnp.where(lanes < E, logits, neg)
    m = jnp.max(logits, axis=-1, keepdims=True)
    ex = jnp.where(lanes < E, jnp.exp(logits - m), 0.0)
    probs = ex / jnp.sum(ex, axis=-1, keepdims=True)
    m1 = jnp.max(probs, axis=-1, keepdims=True)
    a1 = jnp.min(jnp.where(probs >= m1, lanes, E), axis=-1, keepdims=True)
    p2 = jnp.where(lanes == a1, neg, probs)
    m2 = jnp.max(p2, axis=-1, keepdims=True)
    a2 = jnp.min(jnp.where(p2 >= m2, lanes, E), axis=-1, keepdims=True)
    denom = m1 + m2
    w0 = m1 / denom
    w1 = m2 / denom
    col = lanes
    w01_ref[...] = jnp.where(col == 0, w0, 0.0) + jnp.where(col == 1, w1, 0.0)
    j1 = jnp.zeros_like(a1)
    j2 = jnp.zeros_like(a2)
    for e in range(E):
        j1 = j1 + jnp.where(a1 == e, _JMAP[e], 0)
        j2 = j2 + jnp.where(a2 == e, _JMAP[e], 0)
    jp_ref[...] = (jnp.where(col == 0, j1, 0) + jnp.where(col == 1, j2, 0)
                   ).astype(jnp.int32)
    ntok = x.shape[0]
    me = jnp.sum(probs, axis=0) / ntok
    cnt = jnp.sum((lanes == a1).astype(jnp.float32)
                  + (lanes == a2).astype(jnp.float32), axis=0)
    ce = cnt / (ntok * topk)
    aux_ref[...] = jnp.reshape(E * jnp.sum(me * ce), (1, 1))


def _router_call(xf, router_w, router_b, E, topk):
    M, D = xf.shape
    wp = jnp.zeros((D, 128), jnp.float32).at[:, :E].set(router_w)
    bp = jnp.zeros((1, 128), jnp.float32).at[0, :E].set(router_b)
    out = pl.pallas_call(
        functools.partial(_router_body, E=E, topk=topk),
        out_shape=(jax.ShapeDtypeStruct((M, 128), jnp.float32),
                   jax.ShapeDtypeStruct((M, 128), jnp.int32),
                   jax.ShapeDtypeStruct((1, 1), jnp.float32)),
    )(xf, wp, bp)
    return out


def _convfields_body(xm_ref, xb_ref, xp_ref, cw_ref, tbl_ref, *, nsb):
    i = pl.program_id(1)
    xb = xb_ref[0]
    prev = jnp.concatenate([xm_ref[0, -1:], xb[:-1]], axis=0)
    nxt = jnp.concatenate([xb[1:], xp_ref[0, :1]], axis=0)
    rows = jax.lax.broadcasted_iota(jnp.int32, xb.shape, 0)
    prev = jnp.where((i == 0) & (rows == 0), 0.0, prev)
    nxt = jnp.where((i == nsb - 1) & (rows == xb.shape[0] - 1), 0.0, nxt)
    tbl_ref[0, 0] = xb
    for t in range(3):
        c = (prev * cw_ref[t, 0][None, :] + xb * cw_ref[t, 1][None, :]
             + nxt * cw_ref[t, 2][None, :])
        tbl_ref[t + 1, 0] = xb + c


def _convfields_call(x, conv_ws):
    B, S, D = x.shape
    BS = min(512, S)
    nsb = S // BS
    cw = jnp.stack([jnp.transpose(w[:, 0, :], (1, 0)) for w in conv_ws])
    grid = (B, nsb)
    out = pl.pallas_call(
        functools.partial(_convfields_body, nsb=nsb),
        grid=grid,
        in_specs=[
            pl.BlockSpec((1, BS, D), lambda b, i: (b, jnp.maximum(i - 1, 0), 0)),
            pl.BlockSpec((1, BS, D), lambda b, i: (b, i, 0)),
            pl.BlockSpec((1, BS, D), lambda b, i: (b, jnp.minimum(i + 1, nsb - 1), 0)),
            pl.BlockSpec((3, 3, D), lambda b, i: (0, 0, 0)),
        ],
        out_specs=pl.BlockSpec((4, 1, BS, D), lambda b, i: (0, b, i, 0)),
        out_shape=jax.ShapeDtypeStruct((4, B, S, D), jnp.float32),
    )(x, x, x, cw)
    return out.reshape(4, B * S, D)


def _mlp_body(x_ref, w1_ref, b1_ref, w2_ref, b2_ref, gw_ref, gb_ref, o_ref,
              *, gated):
    x = x_ref[...]
    h = jax.lax.dot_general(x, w1_ref[...], (((1,), (0,)), ((), ())),
                            preferred_element_type=jnp.float32, precision=HI)
    h = _gelu(h + b1_ref[...])
    o = jax.lax.dot_general(h, w2_ref[...], (((1,), (0,)), ((), ())),
                            preferred_element_type=jnp.float32,
                            precision=HI) + b2_ref[...]
    if gated:
        g = jax.lax.dot_general(x, gw_ref[...], (((1,), (0,)), ((), ())),
                                preferred_element_type=jnp.float32,
                                precision=HI)
        o *= jax.nn.sigmoid(g[:, :1] + gb_ref[0:1, 0:1])
    o_ref[...] = o


def _mlp_call(xf, w1, b1, w2, b2, gate=None):
    M, D = xf.shape
    F = w1.shape[1]
    N = w2.shape[1]
    BM = min(256, M)
    gated = gate is not None
    if gated:
        gw, gb = gate
        gwp = jnp.zeros((D, 128), jnp.float32).at[:, :1].set(gw)
        gbp = jnp.full((1, 1), gb[0], jnp.float32)
    else:
        gwp = jnp.zeros((1, 128), jnp.float32)
        gbp = jnp.zeros((1, 1), jnp.float32)
    return pl.pallas_call(
        functools.partial(_mlp_body, gated=gated),
        grid=(M // BM,),
        in_specs=[
            pl.BlockSpec((BM, D), lambda m: (m, 0)),
            pl.BlockSpec((D, F), lambda m: (0, 0)),
            pl.BlockSpec((1, F), lambda m: (0, 0)),
            pl.BlockSpec((F, N), lambda m: (0, 0)),
            pl.BlockSpec((1, N), lambda m: (0, 0)),
            pl.BlockSpec(gwp.shape, lambda m: (0, 0)),
            pl.BlockSpec((1, 1), lambda m: (0, 0)),
        ],
        out_specs=pl.BlockSpec((BM, N), lambda m: (m, 0)),
        out_shape=jax.ShapeDtypeStruct((M, N), jnp.float32),
    )(xf, w1, b1.reshape(1, F), w2, b2.reshape(1, N), gwp, gbp)


@functools.lru_cache(maxsize=2)
def _dft_consts(S):
    F = S // 2 + 1
    Fp = ((F + 127) // 128) * 128
    s = np.arange(S)
    f = np.arange(F)
    ang = 2.0 * np.pi * np.outer(f, s) / S
    CS = np.zeros((2 * Fp, S), np.float32)
    CS[:F] = np.cos(ang)
    CS[Fp:Fp + F] = -np.sin(ang)
    cr = np.full(F, 2.0); cr[0] = 1.0; cr[-1] = 1.0
    ci = np.full(F, 2.0); ci[0] = 0.0; ci[-1] = 0.0
    angT = ang.T
    CrCi = np.zeros((2, S, Fp), np.float32)
    CrCi[0, :, :F] = np.cos(angT) * cr / S
    CrCi[1, :, :F] = -np.sin(angT) * ci / S
    return CS, CrCi, Fp


def _matmul_body(a_ref, b_ref, o_ref):
    o_ref[0] = jax.lax.dot_general(
        a_ref[...], b_ref[0], (((1,), (0,)), ((), ())),
        preferred_element_type=jnp.float32, precision=HI)


def _dft_call(x, CS):
    B, S, D = x.shape
    Fp2 = CS.shape[0]
    BM = min(256, Fp2)
    out = pl.pallas_call(
        _matmul_body,
        grid=(B, Fp2 // BM),
        in_specs=[
            pl.BlockSpec((BM, S), lambda b, m: (m, 0)),
            pl.BlockSpec((1, S, D), lambda b, m: (b, 0, 0)),
        ],
        out_specs=pl.BlockSpec((1, BM, D), lambda b, m: (b, m, 0)),
        out_shape=jax.ShapeDtypeStruct((B, Fp2, D), jnp.float32),
    )(CS, x)
    return out.reshape(B, 2, Fp2 // 2, D)


def _fmlp_body(ri_ref, w1_ref, b1_ref, w2_ref, b2_ref, o_ref, *, D):
    re = ri_ref[0, 0]
    im = ri_ref[0, 1]
    h = jax.lax.dot_general(re, w1_ref[:D], (((1,), (0,)), ((), ())),
                            preferred_element_type=jnp.float32, precision=HI)
    h += jax.lax.dot_general(im, w1_ref[D:], (((1,), (0,)), ((), ())),
                             preferred_element_type=jnp.float32, precision=HI)
    h = _gelu(h + b1_ref[...])
    fo = jax.lax.dot_general(h, w2_ref[...], (((1,), (0,)), ((), ())),
                             preferred_element_type=jnp.float32,
                             precision=HI) + b2_ref[...]
    o_ref[0, 0] = fo[:, :D]
    o_ref[1, 0] = fo[:, D:]


def _fmlp_call(RI, w1, b1, w2, b2):
    B, _, Fp, D = RI.shape
    FF = w1.shape[1]
    BM = 192 if Fp % 192 == 0 else min(256, Fp)
    return pl.pallas_call(
        functools.partial(_fmlp_body, D=D),
        grid=(B, Fp // BM),
        in_specs=[
            pl.BlockSpec((1, 2, BM, D), lambda b, m: (b, 0, m, 0)),
            pl.BlockSpec((2 * D, FF), lambda b, m: (0, 0)),
            pl.BlockSpec((1, FF), lambda b, m: (0, 0)),
            pl.BlockSpec((FF, 2 * D), lambda b, m: (0, 0)),
            pl.BlockSpec((1, 2 * D), lambda b, m: (0, 0)),
        ],
        out_specs=pl.BlockSpec((2, 1, BM, D), lambda b, m: (0, b, m, 0)),
        out_shape=jax.ShapeDtypeStruct((2, B, Fp, D), jnp.float32),
    )(RI, w1, b1.reshape(1, FF), w2, b2.reshape(1, 2 * D))


def _irfft_body(c_ref, fo_ref, o_ref):
    o = jax.lax.dot_general(c_ref[0], fo_ref[0, 0],
                            (((1,), (0,)), ((), ())),
                            preferred_element_type=jnp.float32,
                            precision=HI)
    o += jax.lax.dot_general(c_ref[1], fo_ref[1, 0],
                             (((1,), (0,)), ((), ())),
                             preferred_element_type=jnp.float32,
                             precision=HI)
    o_ref[0] = o


def _irfft_call(FO, CrCi):
    _, B, Fp, D = FO.shape
    S = CrCi.shape[1]
    BM = min(256, S)
    return pl.pallas_call(
        _irfft_body,
        grid=(B, S // BM),
        in_specs=[
            pl.BlockSpec((2, BM, Fp), lambda b, s: (0, s, 0)),
            pl.BlockSpec((2, 1, Fp, D), lambda b, s: (0, b, 0, 0)),
        ],
        out_specs=pl.BlockSpec((1, BM, D), lambda b, s: (b, s, 0)),
        out_shape=jax.ShapeDtypeStruct((B, S, D), jnp.float32),
    )(CrCi, FO)


_FLD = [1, 0, 2, 0, 3]


def _dispatch_body(jflat, table, g_hbm, pos_hbm, bexp_hbm, bact_hbm,
                   eidv, gidx, sidx, pidx, buf0, buf1, buf2, bescr, bascr,
                   sem0, sem1, sem2, *, ntok):
    nc = 2
    wid = lax.axis_index("s") * nc + lax.axis_index("c")
    pltpu.sync_copy(jflat, eidv)
    lanes = lax.iota(jnp.int32, 16)

    def count_step(i, accs):
        v = eidv[pl.ds(i * 16, 16)]
        return tuple(a + jnp.where(v == j, 1, 0)
                     for j, a in enumerate(accs))

    def reduce_accs(accs):
        tot = jnp.zeros(16, jnp.int32)
        for j in range(5):
            tot = tot + jnp.where(lanes == j, jnp.sum(accs[j]), 0)
        return tot

    z5 = tuple(jnp.zeros(16, jnp.int32) for _ in range(5))
    totals = reduce_accs(lax.fori_loop(0, _NPAIR // 16, count_step, z5))
    pre = reduce_accs(lax.fori_loop(0, (_CHUNK // 16) * wid, count_step, z5))
    asz = ((totals + (_BMG - 1)) >> 8) << 8
    starts = plsc.cumsum(asz) - asz
    cur = starts + pre

    for v in range(_CHUNK // 16):
        jv = eidv[pl.ds(wid * _CHUNK + v * 16, 16)]
        qv = wid * _CHUNK + v * 16 + lanes
        tv = qv - jnp.where(qv >= ntok, ntok, 0)
        sp = jv < 5
        rank = jnp.zeros(16, jnp.int32)
        basev = jnp.zeros(16, jnp.int32)
        fldv = jnp.zeros(16, jnp.int32)
        for j in range(5):
            mj = jv == j
            cs = plsc.cumsum(mj.astype(jnp.int32))
            rank = rank + jnp.where(mj, cs - 1, 0)
            cj = jnp.sum(jnp.where(lanes == j, cur, 0))
            basev = jnp.where(mj, cj, basev)
            fldv = jnp.where(mj, _FLD[j], fldv)
        slot = basev + rank
        pidx[v // 2, pl.ds((v % 2) * 16, 16)] = jnp.where(sp, slot, 0)
        sidx[v // 2, pl.ds((v % 2) * 16, 16)] = jnp.where(sp, slot, _PT)
        gidx[v // 2, pl.ds((v % 2) * 16, 16)] = jnp.where(
            sp, fldv * ntok + tv, 0)
        for j in range(5):
            c = jnp.sum(jnp.where(jv == j, 1, 0))
            cur = cur + jnp.where(lanes == j, c, 0)

    nchunk = _CHUNK // 32
    bufs = [buf0, buf1, buf2]
    sems = [sem0, sem1, sem2]
    gcp = {}
    scp = {}
    for c in range(3):
        gcp[c] = pltpu.async_copy(table.at[gidx.at[c]], bufs[c], sems[c])
    for c in range(nchunk):
        i = c % 3
        gcp[c].wait()
        scp[c] = pltpu.async_copy(bufs[i], g_hbm.at[sidx.at[c]], sems[i])
        if c + 3 < nchunk:
            scp[c].wait()
            gcp[c + 3] = pltpu.async_copy(table.at[gidx.at[c + 3]],
                                          bufs[i], sems[i])
    for c in range(max(nchunk - 3, 0), nchunk):
        scp[c].wait()

    pltpu.sync_copy(pidx, pos_hbm.at[wid])

    @pl.when(wid == 0)
    def _():
        for g4 in range(4):
            bv = g4 * 16 + lanes
            fr = bv << 8
            be = jnp.zeros(16, jnp.int32)
            ba = jnp.zeros(16, jnp.int32)
            for j in range(5):
                sj = jnp.sum(jnp.where(lanes == j, starts, 0))
                aj = jnp.sum(jnp.where(lanes == j, asz, 0))
                tj = jnp.sum(jnp.where(lanes == j, totals, 0))
                inj = (fr >= sj) & (fr < sj + aj)
                be = jnp.where(inj, j, be)
                ba = jnp.where(inj & (fr < sj + tj), 1, ba)
            bescr[pl.ds(g4 * 16, 16)] = be
            bascr[pl.ds(g4 * 16, 16)] = ba
        pltpu.sync_copy(bescr, bexp_hbm)
        pltpu.sync_copy(bascr, bact_hbm)


def _dispatch_call(jflat, table):
    ntok = table.shape[0] // 4
    D = table.shape[1]
    mesh = plsc.VectorSubcoreMesh(core_axis_name="c", subcore_axis_name="s")
    fn = functools.partial(
        pl.kernel,
        mesh=mesh,
        compiler_params=pltpu.CompilerParams(needs_layout_passes=False),
        out_type=(jax.ShapeDtypeStruct((_PT + 8, D), jnp.float32),
                  jax.ShapeDtypeStruct((_NW, _CHUNK // 32, 32), jnp.int32),
                  jax.ShapeDtypeStruct((64,), jnp.int32),
                  jax.ShapeDtypeStruct((64,), jnp.int32)),
        scratch_types=[
            pltpu.VMEM((_NPAIR,), jnp.int32),
            pltpu.VMEM((_CHUNK // 32, 32), jnp.int32),
            pltpu.VMEM((_CHUNK // 32, 32), jnp.int32),
            pltpu.VMEM((_CHUNK // 32, 32), jnp.int32),
            pltpu.VMEM((32, D), jnp.float32),
            pltpu.VMEM((32, D), jnp.float32),
            pltpu.VMEM((32, D), jnp.float32),
            pltpu.VMEM((64,), jnp.int32),
            pltpu.VMEM((64,), jnp.int32),
            pltpu.SemaphoreType.DMA,
            pltpu.SemaphoreType.DMA,
            pltpu.SemaphoreType.DMA,
        ],
    )(functools.partial(_dispatch_body, ntok=ntok))
    return fn(jflat, table)


def _cgather_body(y_hbm, pos_hbm, out_hbm, pidx, buf0, buf1, buf2,
                  sem0, sem1, sem2):
    nc = 2
    wid = lax.axis_index("s") * nc + lax.axis_index("c")
    pltpu.sync_copy(pos_hbm.at[wid], pidx)
    nchunk = _CHUNK // 32
    bufs = [buf0, buf1, buf2]
    sems = [sem0, sem1, sem2]
    gcp = {}
    scp = {}
    for c in range(min(3, nchunk)):
        gcp[c] = pltpu.async_copy(y_hbm.at[pidx.at[c]], bufs[c], sems[c])
    for c in range(nchunk):
        i = c % 3
        gcp[c].wait()
        scp[c] = pltpu.async_copy(
            bufs[i], out_hbm.at[pl.ds(wid * _CHUNK + c * 32, 32)], sems[i])
        if c + 3 < nchunk:
            scp[c].wait()
            gcp[c + 3] = pltpu.async_copy(y_hbm.at[pidx.at[c + 3]],
                                          bufs[i], sems[i])
    for c in range(max(nchunk - 3, 0), nchunk):
        scp[c].wait()


def _cgather_call(Y, pos):
    D = Y.shape[1]
    mesh = plsc.VectorSubcoreMesh(core_axis_name="c", subcore_axis_name="s")
    fn = functools.partial(
        pl.kernel,
        mesh=mesh,
        compiler_params=pltpu.CompilerParams(needs_layout_passes=False),
        out_type=jax.ShapeDtypeStruct((_NPAIR, D), jnp.float32),
        scratch_types=[
            pltpu.VMEM((_CHUNK // 32, 32), jnp.int32),
            pltpu.VMEM((32, D), jnp.float32),
            pltpu.VMEM((32, D), jnp.float32),
            pltpu.VMEM((32, D), jnp.float32),
            pltpu.SemaphoreType.DMA,
            pltpu.SemaphoreType.DMA,
            pltpu.SemaphoreType.DMA,
        ],
    )(_cgather_body)
    return fn(Y, pos)


def _gmlp_body(bexp_ref, bact_ref, g_ref, w1_ref, b1_ref, w2_ref, b2_ref,
               y_ref):
    b = pl.program_id(0)

    @pl.when(bact_ref[b] == 1)
    def _():
        xg = g_ref[...]
        h = jax.lax.dot_general(xg, w1_ref[0], (((1,), (0,)), ((), ())),
                                preferred_element_type=jnp.float32,
                                precision=HI)
        h = _gelu(h + b1_ref[0])
        y = jax.lax.dot_general(h, w2_ref[0], (((1,), (0,)), ((), ())),
                                preferred_element_type=jnp.float32,
                                precision=HI)
        y_ref[...] = y + b2_ref[0]


def _gmlp_call(G, w1s, b1s, w2s, b2s, bexp, bact):
    D = G.shape[1]
    FF = w1s.shape[2]
    grid_spec = pltpu.PrefetchScalarGridSpec(
        num_scalar_prefetch=2,
        grid=(_NB,),
        in_specs=[
            pl.BlockSpec((_BMG, D), lambda b, be, ba: (b, 0)),
            pl.BlockSpec((1, D, FF), lambda b, be, ba: (be[b], 0, 0)),
            pl.BlockSpec((1, 1, FF), lambda b, be, ba: (be[b], 0, 0)),
            pl.BlockSpec((1, FF, D), lambda b, be, ba: (be[b], 0, 0)),
            pl.BlockSpec((1, 1, D), lambda b, be, ba: (be[b], 0, 0)),
        ],
        out_specs=pl.BlockSpec((_BMG, D), lambda b, be, ba: (b, 0)),
    )
    return pl.pallas_call(
        _gmlp_body,
        grid_spec=grid_spec,
        out_shape=jax.ShapeDtypeStruct((_PT, D), jnp.float32),
    )(bexp, bact, G, w1s, b1s, w2s, b2s)


def _combine_body(base_ref, w_ref, j_ref, g0_ref, g1_ref, *rest):
    eo_refs = rest[:-1]
    o_ref = rest[-1]
    w0 = w_ref[:, 0:1]
    w1 = w_ref[:, 1:2]
    j0 = j_ref[:, 0:1]
    j1 = j_ref[:, 1:2]
    acc = base_ref[...]
    acc = acc + jnp.where(j0 < 5, w0 * g0_ref[...], 0.0)
    acc = acc + jnp.where(j1 < 5, w1 * g1_ref[...], 0.0)
    for jf, eo in enumerate(eo_refs):
        j = 5 + jf
        coef = (jnp.where(j0 == j, w0, 0.0) + jnp.where(j1 == j, w1, 0.0))
        acc = acc + coef * eo[...]
    o_ref[...] = acc


def _combine_call(base, w01, jp, garr, eos):
    M, D = base.shape
    BM = 256
    off = M // BM
    nspec = [pl.BlockSpec((BM, D), lambda m: (m, 0)),
             pl.BlockSpec((BM, 128), lambda m: (m, 0)),
             pl.BlockSpec((BM, 128), lambda m: (m, 0)),
             pl.BlockSpec((BM, D), lambda m: (m, 0)),
             pl.BlockSpec((BM, D), lambda m: (m + off, 0))]
    nspec += [pl.BlockSpec((BM, D), lambda m: (m, 0)) for _ in eos]
    return pl.pallas_call(
        _combine_body,
        grid=(M // BM,),
        in_specs=nspec,
        out_specs=pl.BlockSpec((BM, D), lambda m: (m, 0)),
        out_shape=jax.ShapeDtypeStruct((M, D), jnp.float32),
    )(base, w01, jp, garr, garr, *eos)


def kernel(x, params):
    B, S, D = x.shape
    E = params['router_b'].shape[0]
    xf = x.reshape(B * S, D)

    w01, jp, aux = _router_call(xf, params['router_w'], params['router_b'],
                                E, 2)

    conv_ws = [params['experts'][e]['conv_w'] for e in _SPARSE_EIDS
               if _EXPERT_TYPES[e] == 'conv']
    table = _convfields_call(x, conv_ws)

    jflat = jnp.concatenate([jp[:, 0], jp[:, 1]]).astype(jnp.int32)
    G, pos, bexp, bact = _dispatch_call(jflat, table.reshape(4 * B * S, D))

    base = _mlp_call(xf, params['shared_w1'], params['shared_b1'],
                     params['shared_w2'], params['shared_b2'],
                     gate=(params['gate_w'], params['gate_b']))

    CS_np, CrCi_np, Fp = _dft_consts(S)
    CS = jnp.asarray(CS_np)
    CrCi = jnp.asarray(CrCi_np)
    RI = _dft_call(x, CS)

    w1s = jnp.stack([params['experts'][e]['w1'] for e in _SPARSE_EIDS])
    b1s = jnp.stack([params['experts'][e]['b1'].reshape(1, -1)
                     for e in _SPARSE_EIDS])
    w2s = jnp.stack([params['experts'][e]['w2'] for e in _SPARSE_EIDS])
    b2s = jnp.stack([params['experts'][e]['b2'].reshape(1, -1)
                     for e in _SPARSE_EIDS])
    Y = _gmlp_call(G[:_PT], w1s, b1s, w2s, b2s, bexp, bact)
    garr = _cgather_call(Y, pos)

    f_eos = []
    for e in _FOURIER_EIDS:
        p = params['experts'][e]
        FO = _fmlp_call(RI, p['w1'], p['b1'], p['w2'], p['b2'])
        f_eos.append(_irfft_call(FO, CrCi).reshape(B * S, D))

    out = _combine_call(base, w01, jp, garr, f_eos)
    return out.reshape(B, S, D), aux[0, 0]


# speedup vs baseline: 2.7623x; 1.4295x over previous
import functools
import math

import numpy as np
import jax
import jax.numpy as jnp
from jax import lax
from jax.experimental import pallas as pl
from jax.experimental.pallas import tpu as pltpu
from jax.experimental.pallas import tpu_sc as plsc

HI = None

_BMG = 256
_NW = 32
_NPAIR = 8192
_CHUNK = _NPAIR // _NW
_NB = (_NPAIR + 5 * (_BMG - 1) + _BMG - 1) // _BMG
_PT = _NB * _BMG

_EXPERT_TYPES = ['conv', 'fourier', 'mlp', 'conv', 'fourier', 'mlp', 'conv', 'fourier']
_SPARSE_EIDS = [0, 2, 3, 5, 6]
_FOURIER_EIDS = [1, 4, 7]
_JMAP = [0, 5, 1, 2, 6, 3, 4, 7]


def _gelu(h):
    return h * 0.5 * (1.0 + jax.lax.erf(h / np.float32(np.sqrt(2.0))))


def _router_body(xf, wp, bp, w01_ref, jp_ref, aux_ref, *, E, topk):
    x = xf[...]
    logits = jax.lax.dot_general(x, wp[...], (((1,), (0,)), ((), ())),
                                 preferred_element_type=jnp.float32) + bp[...]
    lanes = jax.lax.broadcasted_iota(jnp.int32, logits.shape, 1)
    neg = jnp.float32(-1e30)
    logits = jnp.where(lanes < E, logits, neg)
    m = jnp.max(logits, axis=-1, keepdims=True)
    ex = jnp.where(lanes < E, jnp.exp(logits - m), 0.0)
    probs = ex / jnp.sum(ex, axis=-1, keepdims=True)
    m1 = jnp.max(probs, axis=-1, keepdims=True)
    a1 = jnp.min(jnp.where(probs >= m1, lanes, E), axis=-1, keepdims=True)
    p2 = jnp.where(lanes == a1, neg, probs)
    m2 = jnp.max(p2, axis=-1, keepdims=True)
    a2 = jnp.min(jnp.where(p2 >= m2, lanes, E), axis=-1, keepdims=True)
    denom = m1 + m2
    w0 = m1 / denom
    w1 = m2 / denom
    col = lanes
    w01_ref[...] = jnp.where(col == 0, w0, 0.0) + jnp.where(col == 1, w1, 0.0)
    j1 = jnp.zeros_like(a1)
    j2 = jnp.zeros_like(a2)
    for e in range(E):
        j1 = j1 + jnp.where(a1 == e, _JMAP[e], 0)
        j2 = j2 + jnp.where(a2 == e, _JMAP[e], 0)
    jp_ref[...] = (jnp.where(col == 0, j1, 0) + jnp.where(col == 1, j2, 0)
                   ).astype(jnp.int32)
    ntok = x.shape[0]
    me = jnp.sum(probs, axis=0) / ntok
    cnt = jnp.sum((lanes == a1).astype(jnp.float32)
                  + (lanes == a2).astype(jnp.float32), axis=0)
    ce = cnt / (ntok * topk)
    aux_ref[...] = jnp.reshape(E * jnp.sum(me * ce), (1, 1))


def _router_call(xf, router_w, router_b, E, topk):
    M, D = xf.shape
    wp = jnp.zeros((D, 128), jnp.float32).at[:, :E].set(router_w)
    bp = jnp.zeros((1, 128), jnp.float32).at[0, :E].set(router_b)
    out = pl.pallas_call(
        functools.partial(_router_body, E=E, topk=topk),
        out_shape=(jax.ShapeDtypeStruct((M, 128), jnp.float32),
                   jax.ShapeDtypeStruct((M, 128), jnp.int32),
                   jax.ShapeDtypeStruct((1, 1), jnp.float32)),
    )(xf, wp, bp)
    return out


def _convfields_body(xm_ref, xb_ref, xp_ref, cw_ref, tbl_ref, *, nsb):
    i = pl.program_id(1)
    xb = xb_ref[0]
    prev = jnp.concatenate([xm_ref[0, -1:], xb[:-1]], axis=0)
    nxt = jnp.concatenate([xb[1:], xp_ref[0, :1]], axis=0)
    rows = jax.lax.broadcasted_iota(jnp.int32, xb.shape, 0)
    prev = jnp.where((i == 0) & (rows == 0), 0.0, prev)
    nxt = jnp.where((i == nsb - 1) & (rows == xb.shape[0] - 1), 0.0, nxt)
    tbl_ref[0, 0] = xb
    for t in range(3):
        c = (prev * cw_ref[t, 0][None, :] + xb * cw_ref[t, 1][None, :]
             + nxt * cw_ref[t, 2][None, :])
        tbl_ref[t + 1, 0] = xb + c


def _convfields_call(x, conv_ws):
    B, S, D = x.shape
    BS = min(512, S)
    nsb = S // BS
    cw = jnp.stack([jnp.transpose(w[:, 0, :], (1, 0)) for w in conv_ws])
    grid = (B, nsb)
    out = pl.pallas_call(
        functools.partial(_convfields_body, nsb=nsb),
        grid=grid,
        in_specs=[
            pl.BlockSpec((1, BS, D), lambda b, i: (b, jnp.maximum(i - 1, 0), 0)),
            pl.BlockSpec((1, BS, D), lambda b, i: (b, i, 0)),
            pl.BlockSpec((1, BS, D), lambda b, i: (b, jnp.minimum(i + 1, nsb - 1), 0)),
            pl.BlockSpec((3, 3, D), lambda b, i: (0, 0, 0)),
        ],
        out_specs=pl.BlockSpec((4, 1, BS, D), lambda b, i: (0, b, i, 0)),
        out_shape=jax.ShapeDtypeStruct((4, B, S, D), jnp.float32),
    )(x, x, x, cw)
    return out.reshape(4, B * S, D)


def _mlp_body(x_ref, w1_ref, b1_ref, w2_ref, b2_ref, gw_ref, gb_ref, o_ref,
              *, gated):
    x = x_ref[...]
    h = jax.lax.dot_general(x, w1_ref[...], (((1,), (0,)), ((), ())),
                            preferred_element_type=jnp.float32, precision=HI)
    h = _gelu(h + b1_ref[...])
    o = jax.lax.dot_general(h, w2_ref[...], (((1,), (0,)), ((), ())),
                            preferred_element_type=jnp.float32,
                            precision=HI) + b2_ref[...]
    if gated:
        g = jax.lax.dot_general(x, gw_ref[...], (((1,), (0,)), ((), ())),
                                preferred_element_type=jnp.float32,
                                precision=HI)
        o *= jax.nn.sigmoid(g[:, :1] + gb_ref[0:1, 0:1])
    o_ref[...] = o


def _mlp_call(xf, w1, b1, w2, b2, gate=None):
    M, D = xf.shape
    F = w1.shape[1]
    N = w2.shape[1]
    BM = min(256, M)
    gated = gate is not None
    if gated:
        gw, gb = gate
        gwp = jnp.zeros((D, 128), jnp.float32).at[:, :1].set(gw)
        gbp = jnp.full((1, 1), gb[0], jnp.float32)
    else:
        gwp = jnp.zeros((1, 128), jnp.float32)
        gbp = jnp.zeros((1, 1), jnp.float32)
    return pl.pallas_call(
        functools.partial(_mlp_body, gated=gated),
        grid=(M // BM,),
        in_specs=[
            pl.BlockSpec((BM, D), lambda m: (m, 0)),
            pl.BlockSpec((D, F), lambda m: (0, 0)),
            pl.BlockSpec((1, F), lambda m: (0, 0)),
            pl.BlockSpec((F, N), lambda m: (0, 0)),
            pl.BlockSpec((1, N), lambda m: (0, 0)),
            pl.BlockSpec(gwp.shape, lambda m: (0, 0)),
            pl.BlockSpec((1, 1), lambda m: (0, 0)),
        ],
        out_specs=pl.BlockSpec((BM, N), lambda m: (m, 0)),
        out_shape=jax.ShapeDtypeStruct((M, N), jnp.float32),
    )(xf, w1, b1.reshape(1, F), w2, b2.reshape(1, N), gwp, gbp)


@functools.lru_cache(maxsize=2)
def _dft_consts(S):
    F = S // 2 + 1
    Fp = ((F + 127) // 128) * 128
    s = np.arange(S)
    f = np.arange(F)
    ang = 2.0 * np.pi * np.outer(f, s) / S
    CS = np.zeros((2 * Fp, S), np.float32)
    CS[:F] = np.cos(ang)
    CS[Fp:Fp + F] = -np.sin(ang)
    cr = np.full(F, 2.0); cr[0] = 1.0; cr[-1] = 1.0
    ci = np.full(F, 2.0); ci[0] = 0.0; ci[-1] = 0.0
    angT = ang.T
    CrCi = np.zeros((2, S, Fp), np.float32)
    CrCi[0, :, :F] = np.cos(angT) * cr / S
    CrCi[1, :, :F] = -np.sin(angT) * ci / S
    return CS, CrCi, Fp


def _matmul_body(a_ref, b_ref, o_ref):
    o_ref[0] = jax.lax.dot_general(
        a_ref[...], b_ref[0], (((1,), (0,)), ((), ())),
        preferred_element_type=jnp.float32, precision=HI)


def _dft_call(x, CS):
    B, S, D = x.shape
    Fp2 = CS.shape[0]
    BM = min(256, Fp2)
    out = pl.pallas_call(
        _matmul_body,
        grid=(B, Fp2 // BM),
        in_specs=[
            pl.BlockSpec((BM, S), lambda b, m: (m, 0)),
            pl.BlockSpec((1, S, D), lambda b, m: (b, 0, 0)),
        ],
        out_specs=pl.BlockSpec((1, BM, D), lambda b, m: (b, m, 0)),
        out_shape=jax.ShapeDtypeStruct((B, Fp2, D), jnp.float32),
    )(CS, x)
    return out.reshape(B, 2, Fp2 // 2, D)


def _fmlp_body(ri_ref, w1_ref, b1_ref, w2_ref, b2_ref, o_ref, *, D):
    re = ri_ref[0, 0]
    im = ri_ref[0, 1]
    h = jax.lax.dot_general(re, w1_ref[:D], (((1,), (0,)), ((), ())),
                            preferred_element_type=jnp.float32, precision=HI)
    h += jax.lax.dot_general(im, w1_ref[D:], (((1,), (0,)), ((), ())),
                             preferred_element_type=jnp.float32, precision=HI)
    h = _gelu(h + b1_ref[...])
    fo = jax.lax.dot_general(h, w2_ref[...], (((1,), (0,)), ((), ())),
                             preferred_element_type=jnp.float32,
                             precision=HI) + b2_ref[...]
    o_ref[0, 0] = fo[:, :D]
    o_ref[1, 0] = fo[:, D:]


def _fmlp_call(RI, w1, b1, w2, b2):
    B, _, Fp, D = RI.shape
    FF = w1.shape[1]
    BM = 192 if Fp % 192 == 0 else min(256, Fp)
    return pl.pallas_call(
        functools.partial(_fmlp_body, D=D),
        grid=(B, Fp // BM),
        in_specs=[
            pl.BlockSpec((1, 2, BM, D), lambda b, m: (b, 0, m, 0)),
            pl.BlockSpec((2 * D, FF), lambda b, m: (0, 0)),
            pl.BlockSpec((1, FF), lambda b, m: (0, 0)),
            pl.BlockSpec((FF, 2 * D), lambda b, m: (0, 0)),
            pl.BlockSpec((1, 2 * D), lambda b, m: (0, 0)),
        ],
        out_specs=pl.BlockSpec((2, 1, BM, D), lambda b, m: (0, b, m, 0)),
        out_shape=jax.ShapeDtypeStruct((2, B, Fp, D), jnp.float32),
    )(RI, w1, b1.reshape(1, FF), w2, b2.reshape(1, 2 * D))


def _irfft_body(c_ref, fo_ref, o_ref):
    o = jax.lax.dot_general(c_ref[0], fo_ref[0, 0],
                            (((1,), (0,)), ((), ())),
                            preferred_element_type=jnp.float32,
                            precision=HI)
    o += jax.lax.dot_general(c_ref[1], fo_ref[1, 0],
                             (((1,), (0,)), ((), ())),
                             preferred_element_type=jnp.float32,
                             precision=HI)
    o_ref[0] = o


def _irfft_call(FO, CrCi):
    _, B, Fp, D = FO.shape
    S = CrCi.shape[1]
    BM = min(256, S)
    return pl.pallas_call(
        _irfft_body,
        grid=(B, S // BM),
        in_specs=[
            pl.BlockSpec((2, BM, Fp), lambda b, s: (0, s, 0)),
            pl.BlockSpec((2, 1, Fp, D), lambda b, s: (0, b, 0, 0)),
        ],
        out_specs=pl.BlockSpec((1, BM, D), lambda b, s: (b, s, 0)),
        out_shape=jax.ShapeDtypeStruct((B, S, D), jnp.float32),
    )(CrCi, FO)


_FLD = [1, 0, 2, 0, 3]


def _dispatch_body(jflat, table, g_hbm, pos_hbm, bexp_hbm, bact_hbm,
                   eidv, gidx, sidx, pidx, buf0, buf1, buf2, bescr, bascr,
                   sem0, sem1, sem2, *, ntok):
    nc = 2
    wid = lax.axis_index("s") * nc + lax.axis_index("c")
    pltpu.sync_copy(jflat, eidv)
    lanes = lax.iota(jnp.int32, 16)

    def count_step(i, accs):
        v = eidv[pl.ds(i * 16, 16)]
        return tuple(a + jnp.where(v == j, 1, 0)
                     for j, a in enumerate(accs))

    def reduce_accs(accs):
        tot = jnp.zeros(16, jnp.int32)
        for j in range(5):
            tot = tot + jnp.where(lanes == j, jnp.sum(accs[j]), 0)
        return tot

    z5 = tuple(jnp.zeros(16, jnp.int32) for _ in range(5))
    totals = reduce_accs(lax.fori_loop(0, _NPAIR // 16, count_step, z5))
    pre = reduce_accs(lax.fori_loop(0, (_CHUNK // 16) * wid, count_step, z5))
    asz = ((totals + (_BMG - 1)) >> 8) << 8
    starts = plsc.cumsum(asz) - asz
    cur = starts + pre

    zero16 = jnp.zeros(16, jnp.int32)
    for v in range(_CHUNK // 16):
        gidx[pl.ds(v * 16, 16)] = zero16
        sidx[pl.ds(v * 16, 16)] = zero16 + _PT

    noff = jnp.int32(0)
    for v in range(_CHUNK // 16):
        jv = eidv[pl.ds(wid * _CHUNK + v * 16, 16)]
        qv = wid * _CHUNK + v * 16 + lanes
        tv = qv - jnp.where(qv >= ntok, ntok, 0)
        sp = jv < 5
        rank = jnp.zeros(16, jnp.int32)
        basev = jnp.zeros(16, jnp.int32)
        fldv = jnp.zeros(16, jnp.int32)
        for j in range(5):
            mj = jv == j
            cs = plsc.cumsum(mj.astype(jnp.int32))
            rank = rank + jnp.where(mj, cs - 1, 0)
            cj = jnp.sum(jnp.where(lanes == j, cur, 0))
            basev = jnp.where(mj, cj, basev)
            fldv = jnp.where(mj, _FLD[j], fldv)
        slot = basev + rank
        pidx[v // 2, pl.ds((v % 2) * 16, 16)] = jnp.where(sp, slot, 0)
        plsc.store_compressed(sidx.at[pl.ds(noff, 16)], slot, mask=sp)
        plsc.store_compressed(gidx.at[pl.ds(noff, 16)],
                              fldv * ntok + tv, mask=sp)
        noff = noff + jnp.sum(jnp.where(sp, 1, 0))
        for j in range(5):
            c = jnp.sum(jnp.where(jv == j, 1, 0))
            cur = cur + jnp.where(lanes == j, c, 0)

    nchunk = _CHUNK // 32
    ncd = (noff + 31) >> 5
    bufs = [buf0, buf1, buf2]
    sems = [sem0, sem1, sem2]
    gcp = [pltpu.make_async_copy(table.at[gidx.at[pl.ds(c * 32, 32)]],
                                 bufs[c % 3], sems[c % 3])
           for c in range(nchunk)]
    scp = [pltpu.make_async_copy(bufs[c % 3],
                                 g_hbm.at[sidx.at[pl.ds(c * 32, 32)]],
                                 sems[c % 3]) for c in range(nchunk)]
    for c in range(3):
        @pl.when(c < ncd)
        def _(c=c):
            gcp[c].start()
    for c in range(nchunk):
        @pl.when(c < ncd)
        def _(c=c):
            gcp[c].wait()
            scp[c].start()
        if c + 3 < nchunk:
            @pl.when(c + 3 < ncd)
            def _(c=c):
                scp[c].wait()
                gcp[c + 3].start()
    for c in range(nchunk):
        @pl.when((c + 3 >= ncd) & (c < ncd))
        def _(c=c):
            scp[c].wait()

    pltpu.sync_copy(pidx, pos_hbm.at[wid])

    @pl.when(wid == 0)
    def _():
        for g4 in range(4):
            bv = g4 * 16 + lanes
            fr = bv << 8
            be = jnp.zeros(16, jnp.int32)
            ba = jnp.zeros(16, jnp.int32)
            for j in range(5):
                sj = jnp.sum(jnp.where(lanes == j, starts, 0))
                aj = jnp.sum(jnp.where(lanes == j, asz, 0))
                tj = jnp.sum(jnp.where(lanes == j, totals, 0))
                inj = (fr >= sj) & (fr < sj + aj)
                be = jnp.where(inj, j, be)
                ba = jnp.where(inj & (fr < sj + tj), 1, ba)
            bescr[pl.ds(g4 * 16, 16)] = be
            bascr[pl.ds(g4 * 16, 16)] = ba
        pltpu.sync_copy(bescr, bexp_hbm)
        pltpu.sync_copy(bascr, bact_hbm)


def _dispatch_call(jflat, table):
    ntok = table.shape[0] // 4
    D = table.shape[1]
    mesh = plsc.VectorSubcoreMesh(core_axis_name="c", subcore_axis_name="s")
    fn = functools.partial(
        pl.kernel,
        mesh=mesh,
        compiler_params=pltpu.CompilerParams(needs_layout_passes=False),
        out_type=(jax.ShapeDtypeStruct((_PT + 8, D), jnp.float32),
                  jax.ShapeDtypeStruct((_NW, _CHUNK // 32, 32), jnp.int32),
                  jax.ShapeDtypeStruct((64,), jnp.int32),
                  jax.ShapeDtypeStruct((64,), jnp.int32)),
        scratch_types=[
            pltpu.VMEM((_NPAIR,), jnp.int32),
            pltpu.VMEM((_CHUNK,), jnp.int32),
            pltpu.VMEM((_CHUNK,), jnp.int32),
            pltpu.VMEM((_CHUNK // 32, 32), jnp.int32),
            pltpu.VMEM((32, D), jnp.float32),
            pltpu.VMEM((32, D), jnp.float32),
            pltpu.VMEM((32, D), jnp.float32),
            pltpu.VMEM((64,), jnp.int32),
            pltpu.VMEM((64,), jnp.int32),
            pltpu.SemaphoreType.DMA,
            pltpu.SemaphoreType.DMA,
            pltpu.SemaphoreType.DMA,
        ],
    )(functools.partial(_dispatch_body, ntok=ntok))
    return fn(jflat, table)


def _cgather_body(y_hbm, pos_hbm, out_hbm, pidx, buf0, buf1, buf2,
                  sem0, sem1, sem2):
    nc = 2
    wid = lax.axis_index("s") * nc + lax.axis_index("c")
    pltpu.sync_copy(pos_hbm.at[wid], pidx)
    nchunk = _CHUNK // 32
    bufs = [buf0, buf1, buf2]
    sems = [sem0, sem1, sem2]
    gcp = {}
    scp = {}
    for c in range(min(3, nchunk)):
        gcp[c] = pltpu.async_copy(y_hbm.at[pidx.at[c]], bufs[c], sems[c])
    for c in range(nchunk):
        i = c % 3
        gcp[c].wait()
        scp[c] = pltpu.async_copy(
            bufs[i], out_hbm.at[pl.ds(wid * _CHUNK + c * 32, 32)], sems[i])
        if c + 3 < nchunk:
            scp[c].wait()
            gcp[c + 3] = pltpu.async_copy(y_hbm.at[pidx.at[c + 3]],
                                          bufs[i], sems[i])
    for c in range(max(nchunk - 3, 0), nchunk):
        scp[c].wait()


def _cgather_call(Y, pos):
    D = Y.shape[1]
    mesh = plsc.VectorSubcoreMesh(core_axis_name="c", subcore_axis_name="s")
    fn = functools.partial(
        pl.kernel,
        mesh=mesh,
        compiler_params=pltpu.CompilerParams(needs_layout_passes=False),
        out_type=jax.ShapeDtypeStruct((_NPAIR, D), jnp.float32),
        scratch_types=[
            pltpu.VMEM((_CHUNK // 32, 32), jnp.int32),
            pltpu.VMEM((32, D), jnp.float32),
            pltpu.VMEM((32, D), jnp.float32),
            pltpu.VMEM((32, D), jnp.float32),
            pltpu.SemaphoreType.DMA,
            pltpu.SemaphoreType.DMA,
            pltpu.SemaphoreType.DMA,
        ],
    )(_cgather_body)
    return fn(Y, pos)


def _gmlp_body(bexp_ref, bact_ref, g_ref, w1_ref, b1_ref, w2_ref, b2_ref,
               y_ref):
    b = pl.program_id(0)

    @pl.when(bact_ref[b] == 1)
    def _():
        xg = g_ref[...]
        h = jax.lax.dot_general(xg, w1_ref[0], (((1,), (0,)), ((), ())),
                                preferred_element_type=jnp.float32,
                                precision=HI)
        h = _gelu(h + b1_ref[0])
        y = jax.lax.dot_general(h, w2_ref[0], (((1,), (0,)), ((), ())),
                                preferred_element_type=jnp.float32,
                                precision=HI)
        y_ref[...] = y + b2_ref[0]


def _gmlp_call(G, w1s, b1s, w2s, b2s, bexp, bact):
    D = G.shape[1]
    FF = w1s.shape[2]
    grid_spec = pltpu.PrefetchScalarGridSpec(
        num_scalar_prefetch=2,
        grid=(_NB,),
        in_specs=[
            pl.BlockSpec((_BMG, D), lambda b, be, ba: (b, 0)),
            pl.BlockSpec((1, D, FF), lambda b, be, ba: (be[b], 0, 0)),
            pl.BlockSpec((1, 1, FF), lambda b, be, ba: (be[b], 0, 0)),
            pl.BlockSpec((1, FF, D), lambda b, be, ba: (be[b], 0, 0)),
            pl.BlockSpec((1, 1, D), lambda b, be, ba: (be[b], 0, 0)),
        ],
        out_specs=pl.BlockSpec((_BMG, D), lambda b, be, ba: (b, 0)),
    )
    return pl.pallas_call(
        _gmlp_body,
        grid_spec=grid_spec,
        out_shape=jax.ShapeDtypeStruct((_PT, D), jnp.float32),
    )(bexp, bact, G, w1s, b1s, w2s, b2s)


def _combine_body(base_ref, w_ref, j_ref, g0_ref, g1_ref, *rest):
    eo_refs = rest[:-1]
    o_ref = rest[-1]
    w0 = w_ref[:, 0:1]
    w1 = w_ref[:, 1:2]
    j0 = j_ref[:, 0:1]
    j1 = j_ref[:, 1:2]
    acc = base_ref[...]
    acc = acc + jnp.where(j0 < 5, w0 * g0_ref[...], 0.0)
    acc = acc + jnp.where(j1 < 5, w1 * g1_ref[...], 0.0)
    for jf, eo in enumerate(eo_refs):
        j = 5 + jf
        coef = (jnp.where(j0 == j, w0, 0.0) + jnp.where(j1 == j, w1, 0.0))
        acc = acc + coef * eo[...]
    o_ref[...] = acc


def _combine_call(base, w01, jp, garr, eos):
    M, D = base.shape
    BM = 256
    off = M // BM
    nspec = [pl.BlockSpec((BM, D), lambda m: (m, 0)),
             pl.BlockSpec((BM, 128), lambda m: (m, 0)),
             pl.BlockSpec((BM, 128), lambda m: (m, 0)),
             pl.BlockSpec((BM, D), lambda m: (m, 0)),
             pl.BlockSpec((BM, D), lambda m: (m + off, 0))]
    nspec += [pl.BlockSpec((BM, D), lambda m: (m, 0)) for _ in eos]
    return pl.pallas_call(
        _combine_body,
        grid=(M // BM,),
        in_specs=nspec,
        out_specs=pl.BlockSpec((BM, D), lambda m: (m, 0)),
        out_shape=jax.ShapeDtypeStruct((M, D), jnp.float32),
    )(base, w01, jp, garr, garr, *eos)


def kernel(x, params):
    B, S, D = x.shape
    E = params['router_b'].shape[0]
    xf = x.reshape(B * S, D)

    w01, jp, aux = _router_call(xf, params['router_w'], params['router_b'],
                                E, 2)

    conv_ws = [params['experts'][e]['conv_w'] for e in _SPARSE_EIDS
               if _EXPERT_TYPES[e] == 'conv']
    table = _convfields_call(x, conv_ws)

    jflat = jnp.concatenate([jp[:, 0], jp[:, 1]]).astype(jnp.int32)
    G, pos, bexp, bact = _dispatch_call(jflat, table.reshape(4 * B * S, D))

    base = _mlp_call(xf, params['shared_w1'], params['shared_b1'],
                     params['shared_w2'], params['shared_b2'],
                     gate=(params['gate_w'], params['gate_b']))

    CS_np, CrCi_np, Fp = _dft_consts(S)
    CS = jnp.asarray(CS_np)
    CrCi = jnp.asarray(CrCi_np)
    RI = _dft_call(x, CS)

    w1s = jnp.stack([params['experts'][e]['w1'] for e in _SPARSE_EIDS])
    b1s = jnp.stack([params['experts'][e]['b1'].reshape(1, -1)
                     for e in _SPARSE_EIDS])
    w2s = jnp.stack([params['experts'][e]['w2'] for e in _SPARSE_EIDS])
    b2s = jnp.stack([params['experts'][e]['b2'].reshape(1, -1)
                     for e in _SPARSE_EIDS])
    Y = _gmlp_call(G[:_PT], w1s, b1s, w2s, b2s, bexp, bact)
    garr = _cgather_call(Y, pos)

    f_eos = []
    for e in _FOURIER_EIDS:
        p = params['experts'][e]
        FO = _fmlp_call(RI, p['w1'], p['b1'], p['w2'], p['b2'])
        f_eos.append(_irfft_call(FO, CrCi).reshape(B * S, D))

    out = _combine_call(base, w01, jp, garr, f_eos)
    return out.reshape(B, S, D), aux[0, 0]


# speedup vs baseline: 3.1401x; 1.1368x over previous
import functools
import math

import numpy as np
import jax
import jax.numpy as jnp
from jax import lax
from jax.experimental import pallas as pl
from jax.experimental.pallas import tpu as pltpu
from jax.experimental.pallas import tpu_sc as plsc

HI = None

_BMG = 256
_NW = 32
_NPAIR = 8192
_CHUNK = _NPAIR // _NW
_NB = (_NPAIR + 5 * (_BMG - 1) + _BMG - 1) // _BMG
_PT = _NB * _BMG

_EXPERT_TYPES = ['conv', 'fourier', 'mlp', 'conv', 'fourier', 'mlp', 'conv', 'fourier']
_SPARSE_EIDS = [0, 2, 3, 5, 6]
_FOURIER_EIDS = [1, 4, 7]
_JMAP = [0, 5, 1, 2, 6, 3, 4, 7]


def _gelu(h):
    return h * 0.5 * (1.0 + jax.lax.erf(h / np.float32(np.sqrt(2.0))))


def _router_body(xf, wp, bp, w01_ref, jp_ref, aux_ref, *, E, topk):
    x = xf[...]
    logits = jax.lax.dot_general(x, wp[...], (((1,), (0,)), ((), ())),
                                 preferred_element_type=jnp.float32) + bp[...]
    lanes = jax.lax.broadcasted_iota(jnp.int32, logits.shape, 1)
    neg = jnp.float32(-1e30)
    logits = jnp.where(lanes < E, logits, neg)
    m = jnp.max(logits, axis=-1, keepdims=True)
    ex = jnp.where(lanes < E, jnp.exp(logits - m), 0.0)
    probs = ex / jnp.sum(ex, axis=-1, keepdims=True)
    m1 = jnp.max(probs, axis=-1, keepdims=True)
    a1 = jnp.min(jnp.where(probs >= m1, lanes, E), axis=-1, keepdims=True)
    p2 = jnp.where(lanes == a1, neg, probs)
    m2 = jnp.max(p2, axis=-1, keepdims=True)
    a2 = jnp.min(jnp.where(p2 >= m2, lanes, E), axis=-1, keepdims=True)
    denom = m1 + m2
    w0 = m1 / denom
    w1 = m2 / denom
    col = lanes
    w01_ref[...] = jnp.where(col == 0, w0, 0.0) + jnp.where(col == 1, w1, 0.0)
    j1 = jnp.zeros_like(a1)
    j2 = jnp.zeros_like(a2)
    for e in range(E):
        j1 = j1 + jnp.where(a1 == e, _JMAP[e], 0)
        j2 = j2 + jnp.where(a2 == e, _JMAP[e], 0)
    jp_ref[...] = (jnp.where(col == 0, j1, 0) + jnp.where(col == 1, j2, 0)
                   ).astype(jnp.int32)
    ntok = x.shape[0]
    me = jnp.sum(probs, axis=0) / ntok
    cnt = jnp.sum((lanes == a1).astype(jnp.float32)
                  + (lanes == a2).astype(jnp.float32), axis=0)
    ce = cnt / (ntok * topk)
    aux_ref[...] = jnp.reshape(E * jnp.sum(me * ce), (1, 1))


def _router_call(xf, router_w, router_b, E, topk):
    M, D = xf.shape
    wp = jnp.zeros((D, 128), jnp.float32).at[:, :E].set(router_w)
    bp = jnp.zeros((1, 128), jnp.float32).at[0, :E].set(router_b)
    out = pl.pallas_call(
        functools.partial(_router_body, E=E, topk=topk),
        out_shape=(jax.ShapeDtypeStruct((M, 128), jnp.float32),
                   jax.ShapeDtypeStruct((M, 128), jnp.int32),
                   jax.ShapeDtypeStruct((1, 1), jnp.float32)),
    )(xf, wp, bp)
    return out


def _convfields_body(xm_ref, xb_ref, xp_ref, cw_ref, tbl_ref, *, nsb):
    i = pl.program_id(1)
    xb = xb_ref[0]
    prev = jnp.concatenate([xm_ref[0, -1:], xb[:-1]], axis=0)
    nxt = jnp.concatenate([xb[1:], xp_ref[0, :1]], axis=0)
    rows = jax.lax.broadcasted_iota(jnp.int32, xb.shape, 0)
    prev = jnp.where((i == 0) & (rows == 0), 0.0, prev)
    nxt = jnp.where((i == nsb - 1) & (rows == xb.shape[0] - 1), 0.0, nxt)
    tbl_ref[0, 0] = xb
    for t in range(3):
        c = (prev * cw_ref[t, 0][None, :] + xb * cw_ref[t, 1][None, :]
             + nxt * cw_ref[t, 2][None, :])
        tbl_ref[t + 1, 0] = xb + c


def _convfields_call(x, conv_ws):
    B, S, D = x.shape
    BS = min(512, S)
    nsb = S // BS
    cw = jnp.stack([jnp.transpose(w[:, 0, :], (1, 0)) for w in conv_ws])
    grid = (B, nsb)
    out = pl.pallas_call(
        functools.partial(_convfields_body, nsb=nsb),
        grid=grid,
        in_specs=[
            pl.BlockSpec((1, BS, D), lambda b, i: (b, jnp.maximum(i - 1, 0), 0)),
            pl.BlockSpec((1, BS, D), lambda b, i: (b, i, 0)),
            pl.BlockSpec((1, BS, D), lambda b, i: (b, jnp.minimum(i + 1, nsb - 1), 0)),
            pl.BlockSpec((3, 3, D), lambda b, i: (0, 0, 0)),
        ],
        out_specs=pl.BlockSpec((4, 1, BS, D), lambda b, i: (0, b, i, 0)),
        out_shape=jax.ShapeDtypeStruct((4, B, S, D), jnp.float32),
    )(x, x, x, cw)
    return out.reshape(4, B * S, D)


def _mlp_body(x_ref, w1_ref, b1_ref, w2_ref, b2_ref, gw_ref, gb_ref, o_ref,
              *, gated):
    x = x_ref[...]
    h = jax.lax.dot_general(x, w1_ref[...], (((1,), (0,)), ((), ())),
                            preferred_element_type=jnp.float32, precision=HI)
    h = _gelu(h + b1_ref[...])
    o = jax.lax.dot_general(h, w2_ref[...], (((1,), (0,)), ((), ())),
                            preferred_element_type=jnp.float32,
                            precision=HI) + b2_ref[...]
    if gated:
        g = jax.lax.dot_general(x, gw_ref[...], (((1,), (0,)), ((), ())),
                                preferred_element_type=jnp.float32,
                                precision=HI)
        o *= jax.nn.sigmoid(g[:, :1] + gb_ref[0:1, 0:1])
    o_ref[...] = o


def _mlp_call(xf, w1, b1, w2, b2, gate=None):
    M, D = xf.shape
    F = w1.shape[1]
    N = w2.shape[1]
    BM = min(256, M)
    gated = gate is not None
    if gated:
        gw, gb = gate
        gwp = jnp.zeros((D, 128), jnp.float32).at[:, :1].set(gw)
        gbp = jnp.full((1, 1), gb[0], jnp.float32)
    else:
        gwp = jnp.zeros((1, 128), jnp.float32)
        gbp = jnp.zeros((1, 1), jnp.float32)
    return pl.pallas_call(
        functools.partial(_mlp_body, gated=gated),
        grid=(M // BM,),
        in_specs=[
            pl.BlockSpec((BM, D), lambda m: (m, 0)),
            pl.BlockSpec((D, F), lambda m: (0, 0)),
            pl.BlockSpec((1, F), lambda m: (0, 0)),
            pl.BlockSpec((F, N), lambda m: (0, 0)),
            pl.BlockSpec((1, N), lambda m: (0, 0)),
            pl.BlockSpec(gwp.shape, lambda m: (0, 0)),
            pl.BlockSpec((1, 1), lambda m: (0, 0)),
        ],
        out_specs=pl.BlockSpec((BM, N), lambda m: (m, 0)),
        out_shape=jax.ShapeDtypeStruct((M, N), jnp.float32),
    )(xf, w1, b1.reshape(1, F), w2, b2.reshape(1, N), gwp, gbp)


@functools.lru_cache(maxsize=2)
def _dft_consts(S):
    F = S // 2 + 1
    Fp = ((F + 127) // 128) * 128
    s = np.arange(S)
    f = np.arange(F)
    ang = 2.0 * np.pi * np.outer(f, s) / S
    CS = np.zeros((2 * Fp, S), np.float32)
    CS[:F] = np.cos(ang)
    CS[Fp:Fp + F] = -np.sin(ang)
    cr = np.full(F, 2.0); cr[0] = 1.0; cr[-1] = 1.0
    ci = np.full(F, 2.0); ci[0] = 0.0; ci[-1] = 0.0
    angT = ang.T
    CrCi = np.zeros((2, S, Fp), np.float32)
    CrCi[0, :, :F] = np.cos(angT) * cr / S
    CrCi[1, :, :F] = -np.sin(angT) * ci / S
    return CS, CrCi, Fp


def _matmul_body(a_ref, b_ref, o_ref):
    o_ref[0] = jax.lax.dot_general(
        a_ref[...], b_ref[0], (((1,), (0,)), ((), ())),
        preferred_element_type=jnp.float32, precision=HI)


def _dft_call(x, CS):
    B, S, D = x.shape
    Fp2 = CS.shape[0]
    BM = min(256, Fp2)
    out = pl.pallas_call(
        _matmul_body,
        grid=(B, Fp2 // BM),
        in_specs=[
            pl.BlockSpec((BM, S), lambda b, m: (m, 0)),
            pl.BlockSpec((1, S, D), lambda b, m: (b, 0, 0)),
        ],
        out_specs=pl.BlockSpec((1, BM, D), lambda b, m: (b, m, 0)),
        out_shape=jax.ShapeDtypeStruct((B, Fp2, D), jnp.float32),
    )(CS, x)
    return out.reshape(B, 2, Fp2 // 2, D)


def _fmlp_body(ri_ref, w1_ref, b1_ref, w2_ref, b2_ref, o_ref, *, D):
    re = ri_ref[0, 0]
    im = ri_ref[0, 1]
    h = jax.lax.dot_general(re, w1_ref[:D], (((1,), (0,)), ((), ())),
                            preferred_element_type=jnp.float32, precision=HI)
    h += jax.lax.dot_general(im, w1_ref[D:], (((1,), (0,)), ((), ())),
                             preferred_element_type=jnp.float32, precision=HI)
    h = _gelu(h + b1_ref[...])
    fo = jax.lax.dot_general(h, w2_ref[...], (((1,), (0,)), ((), ())),
                             preferred_element_type=jnp.float32,
                             precision=HI) + b2_ref[...]
    o_ref[0, 0] = fo[:, :D]
    o_ref[1, 0] = fo[:, D:]


def _fmlp_call(RI, w1, b1, w2, b2):
    B, _, Fp, D = RI.shape
    FF = w1.shape[1]
    BM = 192 if Fp % 192 == 0 else min(256, Fp)
    return pl.pallas_call(
        functools.partial(_fmlp_body, D=D),
        grid=(B, Fp // BM),
        in_specs=[
            pl.BlockSpec((1, 2, BM, D), lambda b, m: (b, 0, m, 0)),
            pl.BlockSpec((2 * D, FF), lambda b, m: (0, 0)),
            pl.BlockSpec((1, FF), lambda b, m: (0, 0)),
            pl.BlockSpec((FF, 2 * D), lambda b, m: (0, 0)),
            pl.BlockSpec((1, 2 * D), lambda b, m: (0, 0)),
        ],
        out_specs=pl.BlockSpec((2, 1, BM, D), lambda b, m: (0, b, m, 0)),
        out_shape=jax.ShapeDtypeStruct((2, B, Fp, D), jnp.float32),
    )(RI, w1, b1.reshape(1, FF), w2, b2.reshape(1, 2 * D))


def _irfft_body(c_ref, fo_ref, o_ref):
    o = jax.lax.dot_general(c_ref[0], fo_ref[0, 0],
                            (((1,), (0,)), ((), ())),
                            preferred_element_type=jnp.float32,
                            precision=HI)
    o += jax.lax.dot_general(c_ref[1], fo_ref[1, 0],
                             (((1,), (0,)), ((), ())),
                             preferred_element_type=jnp.float32,
                             precision=HI)
    o_ref[0] = o


def _irfft_call(FO, CrCi):
    _, B, Fp, D = FO.shape
    S = CrCi.shape[1]
    BM = min(256, S)
    return pl.pallas_call(
        _irfft_body,
        grid=(B, S // BM),
        in_specs=[
            pl.BlockSpec((2, BM, Fp), lambda b, s: (0, s, 0)),
            pl.BlockSpec((2, 1, Fp, D), lambda b, s: (0, b, 0, 0)),
        ],
        out_specs=pl.BlockSpec((1, BM, D), lambda b, s: (b, s, 0)),
        out_shape=jax.ShapeDtypeStruct((B, S, D), jnp.float32),
    )(CrCi, FO)


_FLD = [1, 0, 2, 0, 3]


def _dispatch_body(jflat, table, g_hbm, pos_hbm, bexp_hbm, bact_hbm,
                   eidv, gidx, sidx, pidx, buf0, buf1, buf2, bescr, bascr,
                   sem0, sem1, sem2, *, ntok):
    nc = 2
    wid = lax.axis_index("s") * nc + lax.axis_index("c")
    pltpu.sync_copy(jflat, eidv)
    lanes = lax.iota(jnp.int32, 16)

    def count_step(i, accs):
        v = eidv[pl.ds(i * 16, 16)]
        return tuple(a + jnp.where(v == j, 1, 0)
                     for j, a in enumerate(accs))

    def reduce_accs(accs):
        tot = jnp.zeros(16, jnp.int32)
        for j in range(5):
            tot = tot + jnp.where(lanes == j, jnp.sum(accs[j]), 0)
        return tot

    z5 = tuple(jnp.zeros(16, jnp.int32) for _ in range(5))
    totals = reduce_accs(lax.fori_loop(0, _NPAIR // 16, count_step, z5))
    pre = reduce_accs(lax.fori_loop(0, (_CHUNK // 16) * wid, count_step, z5))
    asz = ((totals + (_BMG - 1)) >> 8) << 8
    starts = plsc.cumsum(asz) - asz
    cur = starts + pre

    zero16 = jnp.zeros(16, jnp.int32)
    for v in range(_CHUNK // 16):
        gidx[pl.ds(v * 16, 16)] = zero16
        sidx[pl.ds(v * 16, 16)] = zero16 + _PT

    noff = jnp.int32(0)
    for v in range(_CHUNK // 16):
        jv = eidv[pl.ds(wid * _CHUNK + v * 16, 16)]
        qv = wid * _CHUNK + v * 16 + lanes
        tv = qv - jnp.where(qv >= ntok, ntok, 0)
        sp = jv < 5
        rank = jnp.zeros(16, jnp.int32)
        basev = jnp.zeros(16, jnp.int32)
        fldv = jnp.zeros(16, jnp.int32)
        for j in range(5):
            mj = jv == j
            cs = plsc.cumsum(mj.astype(jnp.int32))
            rank = rank + jnp.where(mj, cs - 1, 0)
            cj = jnp.sum(jnp.where(lanes == j, cur, 0))
            basev = jnp.where(mj, cj, basev)
            fldv = jnp.where(mj, _FLD[j], fldv)
        slot = basev + rank
        pidx[v // 2, pl.ds((v % 2) * 16, 16)] = jnp.where(sp, slot, 0)
        plsc.store_compressed(sidx.at[pl.ds(noff, 16)], slot, mask=sp)
        plsc.store_compressed(gidx.at[pl.ds(noff, 16)],
                              fldv * ntok + tv, mask=sp)
        noff = noff + jnp.sum(jnp.where(sp, 1, 0))
        for j in range(5):
            c = jnp.sum(jnp.where(jv == j, 1, 0))
            cur = cur + jnp.where(lanes == j, c, 0)

    nchunk = _CHUNK // 32
    ncd = (noff + 31) >> 5
    bufs = [buf0, buf1, buf2]
    sems = [sem0, sem1, sem2]
    gcp = [pltpu.make_async_copy(table.at[gidx.at[pl.ds(c * 32, 32)]],
                                 bufs[c % 3], sems[c % 3])
           for c in range(nchunk)]
    scp = [pltpu.make_async_copy(bufs[c % 3],
                                 g_hbm.at[sidx.at[pl.ds(c * 32, 32)]],
                                 sems[c % 3]) for c in range(nchunk)]
    for c in range(3):
        @pl.when(c < ncd)
        def _(c=c):
            gcp[c].start()
    for c in range(nchunk):
        @pl.when(c < ncd)
        def _(c=c):
            gcp[c].wait()
            scp[c].start()
        if c + 3 < nchunk:
            @pl.when(c + 3 < ncd)
            def _(c=c):
                scp[c].wait()
                gcp[c + 3].start()
    for c in range(nchunk):
        @pl.when((c + 3 >= ncd) & (c < ncd))
        def _(c=c):
            scp[c].wait()

    pltpu.sync_copy(pidx, pos_hbm.at[wid])

    @pl.when(wid == 0)
    def _():
        for g4 in range(4):
            bv = g4 * 16 + lanes
            fr = bv << 8
            be = jnp.zeros(16, jnp.int32)
            ba = jnp.zeros(16, jnp.int32)
            for j in range(5):
                sj = jnp.sum(jnp.where(lanes == j, starts, 0))
                aj = jnp.sum(jnp.where(lanes == j, asz, 0))
                tj = jnp.sum(jnp.where(lanes == j, totals, 0))
                inj = (fr >= sj) & (fr < sj + aj)
                be = jnp.where(inj, j, be)
                ba = jnp.where(inj & (fr < sj + tj), 1, ba)
            bescr[pl.ds(g4 * 16, 16)] = be
            bascr[pl.ds(g4 * 16, 16)] = ba
        pltpu.sync_copy(bescr, bexp_hbm)
        pltpu.sync_copy(bascr, bact_hbm)


def _dispatch_call(jflat, table):
    ntok = table.shape[0] // 4
    D = table.shape[1]
    mesh = plsc.VectorSubcoreMesh(core_axis_name="c", subcore_axis_name="s")
    fn = functools.partial(
        pl.kernel,
        mesh=mesh,
        compiler_params=pltpu.CompilerParams(needs_layout_passes=False),
        out_type=(jax.ShapeDtypeStruct((_PT + 8, D), jnp.float32),
                  jax.ShapeDtypeStruct((_NW, _CHUNK // 32, 32), jnp.int32),
                  jax.ShapeDtypeStruct((64,), jnp.int32),
                  jax.ShapeDtypeStruct((64,), jnp.int32)),
        scratch_types=[
            pltpu.VMEM((_NPAIR,), jnp.int32),
            pltpu.VMEM((_CHUNK,), jnp.int32),
            pltpu.VMEM((_CHUNK,), jnp.int32),
            pltpu.VMEM((_CHUNK // 32, 32), jnp.int32),
            pltpu.VMEM((32, D), jnp.float32),
            pltpu.VMEM((32, D), jnp.float32),
            pltpu.VMEM((32, D), jnp.float32),
            pltpu.VMEM((64,), jnp.int32),
            pltpu.VMEM((64,), jnp.int32),
            pltpu.SemaphoreType.DMA,
            pltpu.SemaphoreType.DMA,
            pltpu.SemaphoreType.DMA,
        ],
    )(functools.partial(_dispatch_body, ntok=ntok))
    return fn(jflat, table)


def _cgather_body(y_hbm, jflat, pos_hbm, out_hbm, pidx, eidc, pc, qc,
                  buf0, buf1, buf2, sem0, sem1, sem2):
    nc = 2
    wid = lax.axis_index("s") * nc + lax.axis_index("c")
    pltpu.sync_copy(pos_hbm.at[wid], pidx)
    pltpu.sync_copy(jflat.at[pl.ds(wid * _CHUNK, _CHUNK)], eidc)
    lanes = lax.iota(jnp.int32, 16)
    zero16 = jnp.zeros(16, jnp.int32)
    for v in range(_CHUNK // 16):
        pc[pl.ds(v * 16, 16)] = zero16
        qc[pl.ds(v * 16, 16)] = zero16 + _NPAIR
    noff = jnp.int32(0)
    for v in range(_CHUNK // 16):
        jv = eidc[pl.ds(v * 16, 16)]
        sp = jv < 5
        pv = pidx[v // 2, pl.ds((v % 2) * 16, 16)]
        qv = wid * _CHUNK + v * 16 + lanes
        plsc.store_compressed(pc.at[pl.ds(noff, 16)], pv, mask=sp)
        plsc.store_compressed(qc.at[pl.ds(noff, 16)], qv, mask=sp)
        noff = noff + jnp.sum(jnp.where(sp, 1, 0))
    nchunk = _CHUNK // 32
    ncd = (noff + 31) >> 5
    bufs = [buf0, buf1, buf2]
    sems = [sem0, sem1, sem2]
    gcp = [pltpu.make_async_copy(y_hbm.at[pc.at[pl.ds(c * 32, 32)]],
                                 bufs[c % 3], sems[c % 3])
           for c in range(nchunk)]
    scp = [pltpu.make_async_copy(bufs[c % 3],
                                 out_hbm.at[qc.at[pl.ds(c * 32, 32)]],
                                 sems[c % 3]) for c in range(nchunk)]
    for c in range(3):
        @pl.when(c < ncd)
        def _(c=c):
            gcp[c].start()
    for c in range(nchunk):
        @pl.when(c < ncd)
        def _(c=c):
            gcp[c].wait()
            scp[c].start()
        if c + 3 < nchunk:
            @pl.when(c + 3 < ncd)
            def _(c=c):
                scp[c].wait()
                gcp[c + 3].start()
    for c in range(nchunk):
        @pl.when((c + 3 >= ncd) & (c < ncd))
        def _(c=c):
            scp[c].wait()


def _cgather_call(Y, jflat, pos):
    D = Y.shape[1]
    mesh = plsc.VectorSubcoreMesh(core_axis_name="c", subcore_axis_name="s")
    fn = functools.partial(
        pl.kernel,
        mesh=mesh,
        compiler_params=pltpu.CompilerParams(needs_layout_passes=False),
        out_type=jax.ShapeDtypeStruct((_NPAIR + 8, D), jnp.float32),
        scratch_types=[
            pltpu.VMEM((_CHUNK // 32, 32), jnp.int32),
            pltpu.VMEM((_CHUNK,), jnp.int32),
            pltpu.VMEM((_CHUNK,), jnp.int32),
            pltpu.VMEM((_CHUNK,), jnp.int32),
            pltpu.VMEM((32, D), jnp.float32),
            pltpu.VMEM((32, D), jnp.float32),
            pltpu.VMEM((32, D), jnp.float32),
            pltpu.SemaphoreType.DMA,
            pltpu.SemaphoreType.DMA,
            pltpu.SemaphoreType.DMA,
        ],
    )(_cgather_body)
    return fn(Y, jflat, pos)


def _gmlp_body(bexp_ref, bact_ref, g_ref, w1_ref, b1_ref, w2_ref, b2_ref,
               y_ref):
    b = pl.program_id(0)

    @pl.when(bact_ref[b] == 1)
    def _():
        xg = g_ref[...]
        h = jax.lax.dot_general(xg, w1_ref[0], (((1,), (0,)), ((), ())),
                                preferred_element_type=jnp.float32,
                                precision=HI)
        h = _gelu(h + b1_ref[0])
        y = jax.lax.dot_general(h, w2_ref[0], (((1,), (0,)), ((), ())),
                                preferred_element_type=jnp.float32,
                                precision=HI)
        y_ref[...] = y + b2_ref[0]


def _gmlp_call(G, w1s, b1s, w2s, b2s, bexp, bact):
    D = G.shape[1]
    FF = w1s.shape[2]
    grid_spec = pltpu.PrefetchScalarGridSpec(
        num_scalar_prefetch=2,
        grid=(_NB,),
        in_specs=[
            pl.BlockSpec((_BMG, D), lambda b, be, ba: (b, 0)),
            pl.BlockSpec((1, D, FF), lambda b, be, ba: (be[b], 0, 0)),
            pl.BlockSpec((1, 1, FF), lambda b, be, ba: (be[b], 0, 0)),
            pl.BlockSpec((1, FF, D), lambda b, be, ba: (be[b], 0, 0)),
            pl.BlockSpec((1, 1, D), lambda b, be, ba: (be[b], 0, 0)),
        ],
        out_specs=pl.BlockSpec((_BMG, D), lambda b, be, ba: (b, 0)),
    )
    return pl.pallas_call(
        _gmlp_body,
        grid_spec=grid_spec,
        out_shape=jax.ShapeDtypeStruct((_PT, D), jnp.float32),
    )(bexp, bact, G, w1s, b1s, w2s, b2s)


def _combine_body(base_ref, w_ref, j_ref, g0_ref, g1_ref, *rest):
    eo_refs = rest[:-1]
    o_ref = rest[-1]
    w0 = w_ref[:, 0:1]
    w1 = w_ref[:, 1:2]
    j0 = j_ref[:, 0:1]
    j1 = j_ref[:, 1:2]
    acc = base_ref[...]
    acc = acc + jnp.where(j0 < 5, w0 * g0_ref[...], 0.0)
    acc = acc + jnp.where(j1 < 5, w1 * g1_ref[...], 0.0)
    for jf, eo in enumerate(eo_refs):
        j = 5 + jf
        coef = (jnp.where(j0 == j, w0, 0.0) + jnp.where(j1 == j, w1, 0.0))
        acc = acc + coef * eo[...]
    o_ref[...] = acc


def _combine_call(base, w01, jp, garr, eos):
    M, D = base.shape
    BM = 256
    off = M // BM
    nspec = [pl.BlockSpec((BM, D), lambda m: (m, 0)),
             pl.BlockSpec((BM, 128), lambda m: (m, 0)),
             pl.BlockSpec((BM, 128), lambda m: (m, 0)),
             pl.BlockSpec((BM, D), lambda m: (m, 0)),
             pl.BlockSpec((BM, D), lambda m: (m + off, 0))]
    nspec += [pl.BlockSpec((BM, D), lambda m: (m, 0)) for _ in eos]
    return pl.pallas_call(
        _combine_body,
        grid=(M // BM,),
        in_specs=nspec,
        out_specs=pl.BlockSpec((BM, D), lambda m: (m, 0)),
        out_shape=jax.ShapeDtypeStruct((M, D), jnp.float32),
    )(base, w01, jp, garr, garr, *eos)


def kernel(x, params):
    B, S, D = x.shape
    E = params['router_b'].shape[0]
    xf = x.reshape(B * S, D)

    w01, jp, aux = _router_call(xf, params['router_w'], params['router_b'],
                                E, 2)

    conv_ws = [params['experts'][e]['conv_w'] for e in _SPARSE_EIDS
               if _EXPERT_TYPES[e] == 'conv']
    table = _convfields_call(x, conv_ws)

    jflat = jnp.concatenate([jp[:, 0], jp[:, 1]]).astype(jnp.int32)
    G, pos, bexp, bact = _dispatch_call(jflat, table.reshape(4 * B * S, D))

    base = _mlp_call(xf, params['shared_w1'], params['shared_b1'],
                     params['shared_w2'], params['shared_b2'],
                     gate=(params['gate_w'], params['gate_b']))

    CS_np, CrCi_np, Fp = _dft_consts(S)
    CS = jnp.asarray(CS_np)
    CrCi = jnp.asarray(CrCi_np)
    RI = _dft_call(x, CS)

    w1s = jnp.stack([params['experts'][e]['w1'] for e in _SPARSE_EIDS])
    b1s = jnp.stack([params['experts'][e]['b1'].reshape(1, -1)
                     for e in _SPARSE_EIDS])
    w2s = jnp.stack([params['experts'][e]['w2'] for e in _SPARSE_EIDS])
    b2s = jnp.stack([params['experts'][e]['b2'].reshape(1, -1)
                     for e in _SPARSE_EIDS])
    Y = _gmlp_call(G[:_PT], w1s, b1s, w2s, b2s, bexp, bact)
    garr = _cgather_call(Y, jflat, pos)

    f_eos = []
    for e in _FOURIER_EIDS:
        p = params['experts'][e]
        FO = _fmlp_call(RI, p['w1'], p['b1'], p['w2'], p['b2'])
        f_eos.append(_irfft_call(FO, CrCi).reshape(B * S, D))

    out = _combine_call(base, w01, jp, garr, f_eos)
    return out.reshape(B, S, D), aux[0, 0]
